# baseline (speedup 1.0000x reference)
import functools

import jax
import jax.numpy as jnp
from jax import lax
from jax.experimental import pallas as pl
from jax.experimental.pallas import tpu as pltpu
from jax.experimental.pallas import tpu_sc as plsc

N_NODES = 10000
N_EDGES = 320000
HID = 128
HALF = 64
L = 4
N_CLASSES = 10
EPS_BN = 1e-5
EPS_DIV = 1e-6

BN_BLK = 2000
BE_BLK = 4000

SC_CORES = 2
SC_SUBCORES = 16
EPW = N_EDGES // SC_SUBCORES
K_CH = 40
N_CHUNKS = EPW // K_CH
N_PAD = 10240
NPW = N_PAD // SC_SUBCORES


def _mm_body(x_ref, w_ref, b_ref, o_ref):
    o_ref[...] = (
        jnp.dot(x_ref[...], w_ref[...], preferred_element_type=jnp.float32)
        + b_ref[...]
    )


def _mm(x, w, b, blk):
    rows = x.shape[0]
    return pl.pallas_call(
        _mm_body,
        grid=(rows // blk,),
        in_specs=[
            pl.BlockSpec((blk, HID), lambda i: (i, 0)),
            pl.BlockSpec((HID, HID), lambda i: (0, 0)),
            pl.BlockSpec((1, HID), lambda i: (0, 0)),
        ],
        out_specs=pl.BlockSpec((blk, HID), lambda i: (i, 0)),
        out_shape=jax.ShapeDtypeStruct((rows, HID), jnp.float32),
    )(x, w, b.reshape(1, HID))


def _node_mm_body(h_ref, w_ref, b_ref, ah_ref, bh_ref, dh_ref, eh_ref):
    h = h_ref[...]
    for t, ref in ((0, ah_ref), (1, bh_ref), (3, dh_ref), (4, eh_ref)):
        ref[...] = (
            jnp.dot(h, w_ref[t], preferred_element_type=jnp.float32) + b_ref[0, t]
        )


def _node_mm(h, w5, b5):
    blk = BN_BLK
    spec = pl.BlockSpec((blk, HID), lambda i: (i, 0))
    shape = jax.ShapeDtypeStruct((N_NODES, HID), jnp.float32)
    return pl.pallas_call(
        _node_mm_body,
        grid=(N_NODES // blk,),
        in_specs=[
            spec,
            pl.BlockSpec((5, HID, HID), lambda i: (0, 0, 0)),
            pl.BlockSpec((1, 5, HID), lambda i: (0, 0, 0)),
        ],
        out_specs=[spec, spec, spec, spec],
        out_shape=[shape, shape, shape, shape],
    )(h, w5, b5.reshape(1, 5, HID))


def _edge_mm_body(e_ref, w_ref, b_ref, o_ref):
    y = jnp.dot(e_ref[...], w_ref[...], preferred_element_type=jnp.float32) + b_ref[...]
    o_ref[0] = y[:, :HALF]
    o_ref[1] = y[:, HALF:]


def _edge_mm(e, w, b):
    blk = BE_BLK
    return pl.pallas_call(
        _edge_mm_body,
        grid=(N_EDGES // blk,),
        in_specs=[
            pl.BlockSpec((blk, HID), lambda i: (i, 0)),
            pl.BlockSpec((HID, HID), lambda i: (0, 0)),
            pl.BlockSpec((1, HID), lambda i: (0, 0)),
        ],
        out_specs=pl.BlockSpec((2, blk, HALF), lambda i: (0, i, 0)),
        out_shape=jax.ShapeDtypeStruct((2, N_EDGES, HALF), jnp.float32),
    )(e, w, b.reshape(1, HID))


def _h_post_body(ah_ref, acc_ref, hin_ref, snn_ref, g_ref, bta_ref, o_ref):
    a0 = acc_ref[0][:N_NODES]
    a1 = acc_ref[1][:N_NODES]
    ssh = jnp.concatenate([a0[:, :HALF], a1[:, :HALF]], axis=1)
    ss = jnp.concatenate([a0[:, HALF:], a1[:, HALF:]], axis=1)
    hn = ah_ref[...] + ssh / (ss + EPS_DIV)
    y = hn * snn_ref[...]
    mean = jnp.mean(y, axis=0, keepdims=True)
    d = y - mean
    var = jnp.mean(d * d, axis=0, keepdims=True)
    yn = g_ref[...] * d * jax.lax.rsqrt(var + EPS_BN) + bta_ref[...]
    o_ref[...] = hin_ref[...] + jnp.maximum(yn, 0.0)


def _h_post(ah, acc, h_in, snn, gamma, beta):
    return pl.pallas_call(
        _h_post_body,
        out_shape=jax.ShapeDtypeStruct((N_NODES, HID), jnp.float32),
    )(ah, acc, h_in, snn, gamma.reshape(1, HID), beta.reshape(1, HID))


def _e_stats_body(en_ref, sne_ref, o_ref, acc_ref):
    i = pl.program_id(0)

    @pl.when(i == 0)
    def _():
        acc_ref[...] = jnp.zeros_like(acc_ref)

    y = jnp.concatenate([en_ref[0], en_ref[1]], axis=1) * sne_ref[...]
    s1 = jnp.sum(y, axis=0)
    s2 = jnp.sum(y * y, axis=0)
    acc_ref[0, :] += s1
    acc_ref[1, :] += s2

    @pl.when(i == pl.num_programs(0) - 1)
    def _():
        o_ref[...] = acc_ref[...]


def _e_stats(enew, sne):
    blk = BE_BLK
    return pl.pallas_call(
        _e_stats_body,
        grid=(N_EDGES // blk,),
        in_specs=[
            pl.BlockSpec((2, blk, HALF), lambda i: (0, i, 0)),
            pl.BlockSpec((blk, 1), lambda i: (i, 0)),
        ],
        out_specs=pl.BlockSpec((2, HID), lambda i: (0, 0)),
        out_shape=jax.ShapeDtypeStruct((2, HID), jnp.float32),
        scratch_shapes=[pltpu.VMEM((2, HID), jnp.float32)],
    )(enew, sne)


def _e_apply_body(ein_ref, en_ref, sne_ref, sc_ref, sh_ref, o_ref):
    y = jnp.concatenate([en_ref[0], en_ref[1]], axis=1) * sne_ref[...]
    yn = y * sc_ref[...] + sh_ref[...]
    o_ref[...] = ein_ref[...] + jnp.maximum(yn, 0.0)


def _e_apply(e_in, enew, sne, scale, shift):
    blk = BE_BLK
    return pl.pallas_call(
        _e_apply_body,
        grid=(N_EDGES // blk,),
        in_specs=[
            pl.BlockSpec((blk, HID), lambda i: (i, 0)),
            pl.BlockSpec((2, blk, HALF), lambda i: (0, i, 0)),
            pl.BlockSpec((blk, 1), lambda i: (i, 0)),
            pl.BlockSpec((1, HID), lambda i: (0, 0)),
            pl.BlockSpec((1, HID), lambda i: (0, 0)),
        ],
        out_specs=pl.BlockSpec((blk, HID), lambda i: (i, 0)),
        out_shape=jax.ShapeDtypeStruct((N_EDGES, HID), jnp.float32),
    )(e_in, enew, sne, scale.reshape(1, HID), shift.reshape(1, HID))


def _sc_mid_body(ce, dh, eh, bh, src, dst, zeros,
                 enew, acc_out,
                 src_v, dst_v, dh_v, eh_v, bh_v, ce_v, sm_v,
                 acc, sem):
    c = lax.axis_index("c")
    s = lax.axis_index("s")
    coff = c * HALF

    row0 = s * NPW
    pltpu.sync_copy(zeros, acc.at[pl.ds(row0, NPW)])
    plsc.subcore_barrier()

    base = s * EPW

    def chunk(j, carry):
        b = base + j * K_CH
        pltpu.sync_copy(src.at[pl.ds(b, K_CH)], src_v)
        pltpu.sync_copy(dst.at[pl.ds(b, K_CH)], dst_v)
        pltpu.async_copy(dh.at[src_v], dh_v, sem).wait()
        pltpu.async_copy(eh.at[dst_v], eh_v, sem).wait()
        pltpu.async_copy(bh.at[src_v], bh_v, sem).wait()
        pltpu.sync_copy(ce.at[c, pl.ds(b, K_CH)], ce_v)

        def row(i, carry2):
            for l in range(HALF // 16):
                sl_t = pl.ds(coff + l * 16, 16)
                sl_h = pl.ds(l * 16, 16)
                x = ce_v[i, sl_h] + dh_v[i, sl_t] + eh_v[i, sl_t]
                sg = 1.0 / (1.0 + jnp.exp(-x))
                ce_v[i, sl_h] = x
                sm_v[i, sl_h] = sg * bh_v[i, sl_t]
                sm_v[i, pl.ds(HALF + l * 16, 16)] = sg
            return carry2

        lax.fori_loop(0, K_CH, row, 0)

        pltpu.sync_copy(ce_v, enew.at[c, pl.ds(b, K_CH)])
        pltpu.sync_copy(sm_v, acc.at[dst_v], add=True)
        return carry

    lax.fori_loop(0, N_CHUNKS, chunk, 0)
    plsc.subcore_barrier()

    pltpu.sync_copy(acc.at[pl.ds(row0, NPW)], acc_out.at[c, pl.ds(row0, NPW)])


_sc_mid = pl.kernel(
    _sc_mid_body,
    out_type=(
        jax.ShapeDtypeStruct((2, N_EDGES, HALF), jnp.float32),
        jax.ShapeDtypeStruct((2, N_PAD, HID), jnp.float32),
    ),
    mesh=plsc.VectorSubcoreMesh(core_axis_name="c", subcore_axis_name="s"),
    scratch_types=[
        pltpu.VMEM((K_CH,), jnp.int32),
        pltpu.VMEM((K_CH,), jnp.int32),
        pltpu.VMEM((K_CH, HID), jnp.float32),
        pltpu.VMEM((K_CH, HID), jnp.float32),
        pltpu.VMEM((K_CH, HID), jnp.float32),
        pltpu.VMEM((K_CH, HALF), jnp.float32),
        pltpu.VMEM((K_CH, HID), jnp.float32),
        pltpu.VMEM_SHARED((N_PAD, HID), jnp.float32),
        pltpu.SemaphoreType.DMA,
    ],
)


def kernel(h, e, edge_index, snorm_n, snorm_e, emb_h_W, emb_h_b, emb_e_W,
           emb_e_b, lin_W, lin_b, bn_scale, bn_bias, mlp_W, mlp_b):
    src = edge_index[0]
    dst = edge_index[1]
    snn = snorm_n.reshape(N_NODES, 1)
    sne = snorm_e.reshape(N_EDGES, 1)
    zeros_stripe = jnp.zeros((NPW, HID), jnp.float32)

    h = _mm(h, emb_h_W, emb_h_b, BN_BLK)
    e = _mm(e, emb_e_W, emb_e_b, BE_BLK)

    for l in range(L):
        h_in, e_in = h, e
        ah, bh, dh, eh = _node_mm(h, lin_W[l], lin_b[l])
        ce2 = _edge_mm(e, lin_W[l, 2], lin_b[l, 2])

        enew2, acc2 = _sc_mid(ce2, dh, eh, bh, src, dst, zeros_stripe)

        h = _h_post(ah, acc2, h_in, snn, bn_scale[l, 0], bn_bias[l, 0])

        stats = _e_stats(enew2, sne)
        mean = stats[0] / N_EDGES
        var = stats[1] / N_EDGES - mean * mean
        scale = bn_scale[l, 1] * jax.lax.rsqrt(var + EPS_BN)
        shift = bn_bias[l, 1] - mean * scale
        e = _e_apply(e_in, enew2, sne, scale, shift)

    w_pad = jnp.zeros((HID, HID), jnp.float32).at[:, :N_CLASSES].set(mlp_W)
    b_pad = jnp.zeros((HID,), jnp.float32).at[:N_CLASSES].set(mlp_b)
    out = _mm(h, w_pad, b_pad, BN_BLK)
    return out[:, :N_CLASSES]


# speedup vs baseline: 1.9028x; 1.9028x over previous
import functools

import jax
import jax.numpy as jnp
from jax import lax
from jax.experimental import pallas as pl
from jax.experimental.pallas import tpu as pltpu
from jax.experimental.pallas import tpu_sc as plsc

N_NODES = 10000
N_EDGES = 320000
HID = 128
HALF = 64
L = 4
N_CLASSES = 10
EPS_BN = 1e-5
EPS_DIV = 1e-6

BN_BLK = 2000
BE_BLK = 4000

SC_CORES = 2
SC_SUBCORES = 16
EPW = N_EDGES // SC_SUBCORES
K_CH = 40
N_CHUNKS = EPW // K_CH
N_PAD = 10240
NPW = N_PAD // SC_SUBCORES


def _mm_body(x_ref, w_ref, b_ref, o_ref):
    o_ref[...] = (
        jnp.dot(x_ref[...], w_ref[...], preferred_element_type=jnp.float32)
        + b_ref[...]
    )


def _mm(x, w, b, blk):
    rows = x.shape[0]
    return pl.pallas_call(
        _mm_body,
        grid=(rows // blk,),
        in_specs=[
            pl.BlockSpec((blk, HID), lambda i: (i, 0)),
            pl.BlockSpec((HID, HID), lambda i: (0, 0)),
            pl.BlockSpec((1, HID), lambda i: (0, 0)),
        ],
        out_specs=pl.BlockSpec((blk, HID), lambda i: (i, 0)),
        out_shape=jax.ShapeDtypeStruct((rows, HID), jnp.float32),
    )(x, w, b.reshape(1, HID))


def _node_mm_body(h_ref, w_ref, b_ref, ah_ref, t1_ref, eh_ref):
    h = h_ref[...]
    ah_ref[...] = jnp.dot(h, w_ref[0], preferred_element_type=jnp.float32) + b_ref[0, 0]
    eh_ref[...] = jnp.dot(h, w_ref[4], preferred_element_type=jnp.float32) + b_ref[0, 4]
    bh = jnp.dot(h, w_ref[1], preferred_element_type=jnp.float32) + b_ref[0, 1]
    dh = jnp.dot(h, w_ref[3], preferred_element_type=jnp.float32) + b_ref[0, 3]
    t1_ref[0] = jnp.concatenate([dh[:, :HALF], bh[:, :HALF]], axis=1)
    t1_ref[1] = jnp.concatenate([dh[:, HALF:], bh[:, HALF:]], axis=1)


def _node_mm(h, w5, b5):
    blk = BN_BLK
    spec = pl.BlockSpec((blk, HID), lambda i: (i, 0))
    shape = jax.ShapeDtypeStruct((N_NODES, HID), jnp.float32)
    return pl.pallas_call(
        _node_mm_body,
        grid=(N_NODES // blk,),
        in_specs=[
            spec,
            pl.BlockSpec((5, HID, HID), lambda i: (0, 0, 0)),
            pl.BlockSpec((1, 5, HID), lambda i: (0, 0, 0)),
        ],
        out_specs=[
            spec,
            pl.BlockSpec((2, blk, HID), lambda i: (0, i, 0)),
            spec,
        ],
        out_shape=[
            shape,
            jax.ShapeDtypeStruct((2, N_NODES, HID), jnp.float32),
            shape,
        ],
    )(h, w5, b5.reshape(1, 5, HID))


def _edge_mm_body(e_ref, w_ref, b_ref, o_ref):
    y = jnp.dot(e_ref[...], w_ref[...], preferred_element_type=jnp.float32) + b_ref[...]
    o_ref[0] = y[:, :HALF]
    o_ref[1] = y[:, HALF:]


def _edge_mm(e, w, b):
    blk = BE_BLK
    return pl.pallas_call(
        _edge_mm_body,
        grid=(N_EDGES // blk,),
        in_specs=[
            pl.BlockSpec((blk, HID), lambda i: (i, 0)),
            pl.BlockSpec((HID, HID), lambda i: (0, 0)),
            pl.BlockSpec((1, HID), lambda i: (0, 0)),
        ],
        out_specs=pl.BlockSpec((2, blk, HALF), lambda i: (0, i, 0)),
        out_shape=jax.ShapeDtypeStruct((2, N_EDGES, HALF), jnp.float32),
    )(e, w, b.reshape(1, HID))


def _h_post_body(ah_ref, acc_ref, hin_ref, snn_ref, g_ref, bta_ref, o_ref):
    a0 = acc_ref[0][:N_NODES]
    a1 = acc_ref[1][:N_NODES]
    ssh = jnp.concatenate([a0[:, :HALF], a1[:, :HALF]], axis=1)
    ss = jnp.concatenate([a0[:, HALF:], a1[:, HALF:]], axis=1)
    hn = ah_ref[...] + ssh / (ss + EPS_DIV)
    y = hn * snn_ref[...]
    mean = jnp.mean(y, axis=0, keepdims=True)
    d = y - mean
    var = jnp.mean(d * d, axis=0, keepdims=True)
    yn = g_ref[...] * d * jax.lax.rsqrt(var + EPS_BN) + bta_ref[...]
    o_ref[...] = hin_ref[...] + jnp.maximum(yn, 0.0)


def _h_post(ah, acc, h_in, snn, gamma, beta):
    return pl.pallas_call(
        _h_post_body,
        out_shape=jax.ShapeDtypeStruct((N_NODES, HID), jnp.float32),
    )(ah, acc, h_in, snn, gamma.reshape(1, HID), beta.reshape(1, HID))


def _e_stats_body(en_ref, sne_ref, o_ref, acc_ref):
    i = pl.program_id(0)

    @pl.when(i == 0)
    def _():
        acc_ref[...] = jnp.zeros_like(acc_ref)

    y = jnp.concatenate([en_ref[0], en_ref[1]], axis=1) * sne_ref[...]
    s1 = jnp.sum(y, axis=0)
    s2 = jnp.sum(y * y, axis=0)
    acc_ref[0, :] += s1
    acc_ref[1, :] += s2

    @pl.when(i == pl.num_programs(0) - 1)
    def _():
        o_ref[...] = acc_ref[...]


def _e_stats(enew, sne):
    blk = BE_BLK
    return pl.pallas_call(
        _e_stats_body,
        grid=(N_EDGES // blk,),
        in_specs=[
            pl.BlockSpec((2, blk, HALF), lambda i: (0, i, 0)),
            pl.BlockSpec((blk, 1), lambda i: (i, 0)),
        ],
        out_specs=pl.BlockSpec((2, HID), lambda i: (0, 0)),
        out_shape=jax.ShapeDtypeStruct((2, HID), jnp.float32),
        scratch_shapes=[pltpu.VMEM((2, HID), jnp.float32)],
    )(enew, sne)


def _e_apply_body(ein_ref, en_ref, sne_ref, sc_ref, sh_ref, o_ref):
    y = jnp.concatenate([en_ref[0], en_ref[1]], axis=1) * sne_ref[...]
    yn = y * sc_ref[...] + sh_ref[...]
    o_ref[...] = ein_ref[...] + jnp.maximum(yn, 0.0)


def _e_apply(e_in, enew, sne, scale, shift):
    blk = BE_BLK
    return pl.pallas_call(
        _e_apply_body,
        grid=(N_EDGES // blk,),
        in_specs=[
            pl.BlockSpec((blk, HID), lambda i: (i, 0)),
            pl.BlockSpec((2, blk, HALF), lambda i: (0, i, 0)),
            pl.BlockSpec((blk, 1), lambda i: (i, 0)),
            pl.BlockSpec((1, HID), lambda i: (0, 0)),
            pl.BlockSpec((1, HID), lambda i: (0, 0)),
        ],
        out_specs=pl.BlockSpec((blk, HID), lambda i: (i, 0)),
        out_shape=jax.ShapeDtypeStruct((N_EDGES, HID), jnp.float32),
    )(e_in, enew, sne, scale.reshape(1, HID), shift.reshape(1, HID))


SUP = 10
NSUP = N_CHUNKS // SUP


def _sc_mid_body(ce, t1, eh, src4, dst4, zeros,
                 enew, acc_out,
                 src_sv, dst_sv, de0, de1, ev0, ev1, cv0, cv1, sm0, sm1,
                 acc, sem_g0, sem_g1, sem_o0, sem_o1):
    c = lax.axis_index("c")
    s = lax.axis_index("s")
    coff = c * HALF
    de = (de0, de1)
    ev = (ev0, ev1)
    cv = (cv0, cv1)
    sm = (sm0, sm1)
    sem_g = (sem_g0, sem_g1)
    sem_o = (sem_o0, sem_o1)

    row0 = s * NPW
    pltpu.sync_copy(zeros, acc.at[pl.ds(row0, NPW)])
    plsc.subcore_barrier()

    base = s * EPW

    def load_idx(u):
        us = lax.rem(u, 2)
        pltpu.sync_copy(src4.at[s, u], src_sv.at[us])
        pltpu.sync_copy(dst4.at[s, u], dst_sv.at[us])

    def gather_refs(t, p):
        u = t // SUP
        us = lax.rem(u, 2)
        jj = lax.rem(t, SUP)
        b = base + t * K_CH
        return (
            (t1.at[c].at[src_sv.at[us, jj]], de[p]),
            (eh.at[dst_sv.at[us, jj]], ev[p]),
            (ce.at[c, pl.ds(b, K_CH)], cv[p]),
        )

    def issue_chunk(t, p):
        for sref, dref in gather_refs(t, p):
            pltpu.async_copy(sref, dref, sem_g[p])

    def wait_chunk(t, p):
        for sref, dref in gather_refs(t, p):
            pltpu.make_async_copy(sref, dref, sem_g[p]).wait()

    def issue_out(t, p):
        b = base + t * K_CH
        pltpu.async_copy(cv[p], enew.at[c, pl.ds(b, K_CH)], sem_o[p])

    def wait_out(t, p):
        b = base + t * K_CH
        pltpu.make_async_copy(cv[p], enew.at[c, pl.ds(b, K_CH)], sem_o[p]).wait()

    def scatter(t, p):
        u = t // SUP
        us = lax.rem(u, 2)
        jj = lax.rem(t, SUP)
        pltpu.sync_copy(sm[p], acc.at[dst_sv.at[us, jj]], add=True)

    def compute(p):
        def row(i, carry):
            for l in range(HALF // 16):
                sl_h = pl.ds(l * 16, 16)
                sl_e = pl.ds(coff + l * 16, 16)
                sl_b = pl.ds(HALF + l * 16, 16)
                x = cv[p][i, sl_h] + de[p][i, sl_h] + ev[p][i, sl_e]
                sg = 1.0 / (1.0 + jnp.exp(-x))
                cv[p][i, sl_h] = x
                sm[p][i, sl_h] = sg * de[p][i, sl_b]
                sm[p][i, sl_b] = sg
            return carry

        lax.fori_loop(0, K_CH, row, 0)

    load_idx(0)
    issue_chunk(0, 0)

    def pair(i, carry):
        for par in range(2):
            t = 2 * i + par
            p = par
            q = 1 - par
            tn = t + 1
            wait_chunk(t, p)

            @pl.when(t >= 2)
            def _():
                wait_out(t - 2, p)

            @pl.when(jnp.logical_and(tn < N_CHUNKS, lax.rem(tn, SUP) == 0))
            def _():
                load_idx(tn // SUP)

            @pl.when(tn < N_CHUNKS)
            def _():
                issue_chunk(tn, q)

            compute(p)
            issue_out(t, p)
            scatter(t, p)
        return carry

    lax.fori_loop(0, N_CHUNKS // 2, pair, 0)
    wait_out(N_CHUNKS - 2, 0)
    wait_out(N_CHUNKS - 1, 1)
    plsc.subcore_barrier()

    pltpu.sync_copy(acc.at[pl.ds(row0, NPW)], acc_out.at[c, pl.ds(row0, NPW)])


_sc_mid = pl.kernel(
    _sc_mid_body,
    out_type=(
        jax.ShapeDtypeStruct((2, N_EDGES, HALF), jnp.float32),
        jax.ShapeDtypeStruct((2, N_PAD, HID), jnp.float32),
    ),
    mesh=plsc.VectorSubcoreMesh(core_axis_name="c", subcore_axis_name="s"),
    scratch_types=[
        pltpu.VMEM((2, SUP, K_CH), jnp.int32),
        pltpu.VMEM((2, SUP, K_CH), jnp.int32),
        pltpu.VMEM((K_CH, HID), jnp.float32),
        pltpu.VMEM((K_CH, HID), jnp.float32),
        pltpu.VMEM((K_CH, HID), jnp.float32),
        pltpu.VMEM((K_CH, HID), jnp.float32),
        pltpu.VMEM((K_CH, HALF), jnp.float32),
        pltpu.VMEM((K_CH, HALF), jnp.float32),
        pltpu.VMEM((K_CH, HID), jnp.float32),
        pltpu.VMEM((K_CH, HID), jnp.float32),
        pltpu.VMEM_SHARED((N_PAD, HID), jnp.float32),
        pltpu.SemaphoreType.DMA,
        pltpu.SemaphoreType.DMA,
        pltpu.SemaphoreType.DMA,
        pltpu.SemaphoreType.DMA,
    ],
)


def kernel(h, e, edge_index, snorm_n, snorm_e, emb_h_W, emb_h_b, emb_e_W,
           emb_e_b, lin_W, lin_b, bn_scale, bn_bias, mlp_W, mlp_b):
    src4 = edge_index[0].reshape(SC_SUBCORES, NSUP, SUP, K_CH)
    dst4 = edge_index[1].reshape(SC_SUBCORES, NSUP, SUP, K_CH)
    snn = snorm_n.reshape(N_NODES, 1)
    sne = snorm_e.reshape(N_EDGES, 1)
    zeros_stripe = jnp.zeros((NPW, HID), jnp.float32)

    h = _mm(h, emb_h_W, emb_h_b, BN_BLK)
    e = _mm(e, emb_e_W, emb_e_b, BE_BLK)

    for l in range(L):
        h_in, e_in = h, e
        ah, t1, eh = _node_mm(h, lin_W[l], lin_b[l])
        ce2 = _edge_mm(e, lin_W[l, 2], lin_b[l, 2])

        enew2, acc2 = _sc_mid(ce2, t1, eh, src4, dst4, zeros_stripe)

        h = _h_post(ah, acc2, h_in, snn, bn_scale[l, 0], bn_bias[l, 0])

        stats = _e_stats(enew2, sne)
        mean = stats[0] / N_EDGES
        var = stats[1] / N_EDGES - mean * mean
        scale = bn_scale[l, 1] * jax.lax.rsqrt(var + EPS_BN)
        shift = bn_bias[l, 1] - mean * scale
        e = _e_apply(e_in, enew2, sne, scale, shift)

    w_pad = jnp.zeros((HID, HID), jnp.float32).at[:, :N_CLASSES].set(mlp_W)
    b_pad = jnp.zeros((HID,), jnp.float32).at[:N_CLASSES].set(mlp_b)
    out = _mm(h, w_pad, b_pad, BN_BLK)
    return out[:, :N_CLASSES]


# speedup vs baseline: 1.9259x; 1.0121x over previous
import functools

import jax
import jax.numpy as jnp
from jax import lax
from jax.experimental import pallas as pl
from jax.experimental.pallas import tpu as pltpu
from jax.experimental.pallas import tpu_sc as plsc

N_NODES = 10000
N_EDGES = 320000
HID = 128
HALF = 64
L = 4
N_CLASSES = 10
EPS_BN = 1e-5
EPS_DIV = 1e-6

BN_BLK = 2000
BE_BLK = 4000

SC_CORES = 2
SC_SUBCORES = 16
EPW = N_EDGES // SC_SUBCORES
K_CH = 40
N_CHUNKS = EPW // K_CH
N_PAD = 10240
NPW = N_PAD // SC_SUBCORES


def _mm_body(x_ref, w_ref, b_ref, o_ref):
    o_ref[...] = (
        jnp.dot(x_ref[...], w_ref[...], preferred_element_type=jnp.float32)
        + b_ref[...]
    )


def _mm(x, w, b, blk):
    rows = x.shape[0]
    return pl.pallas_call(
        _mm_body,
        grid=(rows // blk,),
        in_specs=[
            pl.BlockSpec((blk, HID), lambda i: (i, 0)),
            pl.BlockSpec((HID, HID), lambda i: (0, 0)),
            pl.BlockSpec((1, HID), lambda i: (0, 0)),
        ],
        out_specs=pl.BlockSpec((blk, HID), lambda i: (i, 0)),
        out_shape=jax.ShapeDtypeStruct((rows, HID), jnp.float32),
    )(x, w, b.reshape(1, HID))


def _node_mm_body(h_ref, w_ref, b_ref, ah_ref, t1_ref, eh_ref):
    h = h_ref[...]
    ah_ref[...] = jnp.dot(h, w_ref[0], preferred_element_type=jnp.float32) + b_ref[0, 0]
    eh_ref[...] = jnp.dot(h, w_ref[4], preferred_element_type=jnp.float32) + b_ref[0, 4]
    bh = jnp.dot(h, w_ref[1], preferred_element_type=jnp.float32) + b_ref[0, 1]
    dh = jnp.dot(h, w_ref[3], preferred_element_type=jnp.float32) + b_ref[0, 3]
    t1_ref[0] = jnp.concatenate([dh[:, :HALF], bh[:, :HALF]], axis=1)
    t1_ref[1] = jnp.concatenate([dh[:, HALF:], bh[:, HALF:]], axis=1)


def _node_mm(h, w5, b5):
    blk = BN_BLK
    spec = pl.BlockSpec((blk, HID), lambda i: (i, 0))
    shape = jax.ShapeDtypeStruct((N_NODES, HID), jnp.float32)
    return pl.pallas_call(
        _node_mm_body,
        grid=(N_NODES // blk,),
        in_specs=[
            spec,
            pl.BlockSpec((5, HID, HID), lambda i: (0, 0, 0)),
            pl.BlockSpec((1, 5, HID), lambda i: (0, 0, 0)),
        ],
        out_specs=[
            spec,
            pl.BlockSpec((2, blk, HID), lambda i: (0, i, 0)),
            spec,
        ],
        out_shape=[
            shape,
            jax.ShapeDtypeStruct((2, N_NODES, HID), jnp.float32),
            shape,
        ],
    )(h, w5, b5.reshape(1, 5, HID))


def _edge_mm_body(e_ref, w_ref, b_ref, o_ref):
    y = jnp.dot(e_ref[...], w_ref[...], preferred_element_type=jnp.float32) + b_ref[...]
    o_ref[0] = y[:, :HALF]
    o_ref[1] = y[:, HALF:]


def _edge_mm(e, w, b):
    blk = BE_BLK
    return pl.pallas_call(
        _edge_mm_body,
        grid=(N_EDGES // blk,),
        in_specs=[
            pl.BlockSpec((blk, HID), lambda i: (i, 0)),
            pl.BlockSpec((HID, HID), lambda i: (0, 0)),
            pl.BlockSpec((1, HID), lambda i: (0, 0)),
        ],
        out_specs=pl.BlockSpec((2, blk, HALF), lambda i: (0, i, 0)),
        out_shape=jax.ShapeDtypeStruct((2, N_EDGES, HALF), jnp.float32),
    )(e, w, b.reshape(1, HID))


def _h_post_body(ah_ref, acc_ref, hin_ref, snn_ref, g_ref, bta_ref, o_ref):
    a0 = acc_ref[0][:N_NODES]
    a1 = acc_ref[1][:N_NODES]
    ssh = jnp.concatenate([a0[:, :HALF], a1[:, :HALF]], axis=1)
    ss = jnp.concatenate([a0[:, HALF:], a1[:, HALF:]], axis=1)
    hn = ah_ref[...] + ssh / (ss + EPS_DIV)
    y = hn * snn_ref[...]
    mean = jnp.mean(y, axis=0, keepdims=True)
    d = y - mean
    var = jnp.mean(d * d, axis=0, keepdims=True)
    yn = g_ref[...] * d * jax.lax.rsqrt(var + EPS_BN) + bta_ref[...]
    o_ref[...] = hin_ref[...] + jnp.maximum(yn, 0.0)


def _h_post(ah, acc, h_in, snn, gamma, beta):
    return pl.pallas_call(
        _h_post_body,
        out_shape=jax.ShapeDtypeStruct((N_NODES, HID), jnp.float32),
    )(ah, acc, h_in, snn, gamma.reshape(1, HID), beta.reshape(1, HID))


def _e_stats_body(en_ref, sne_ref, o_ref, acc_ref):
    i = pl.program_id(0)

    @pl.when(i == 0)
    def _():
        acc_ref[...] = jnp.zeros_like(acc_ref)

    y = jnp.concatenate([en_ref[0], en_ref[1]], axis=1) * sne_ref[...]
    s1 = jnp.sum(y, axis=0)
    s2 = jnp.sum(y * y, axis=0)
    acc_ref[0, :] += s1
    acc_ref[1, :] += s2

    @pl.when(i == pl.num_programs(0) - 1)
    def _():
        o_ref[...] = acc_ref[...]


def _e_stats(enew, sne):
    blk = BE_BLK
    return pl.pallas_call(
        _e_stats_body,
        grid=(N_EDGES // blk,),
        in_specs=[
            pl.BlockSpec((2, blk, HALF), lambda i: (0, i, 0)),
            pl.BlockSpec((blk, 1), lambda i: (i, 0)),
        ],
        out_specs=pl.BlockSpec((2, HID), lambda i: (0, 0)),
        out_shape=jax.ShapeDtypeStruct((2, HID), jnp.float32),
        scratch_shapes=[pltpu.VMEM((2, HID), jnp.float32)],
    )(enew, sne)


def _e_apply_body(ein_ref, en_ref, sne_ref, sc_ref, sh_ref, o_ref):
    y = jnp.concatenate([en_ref[0], en_ref[1]], axis=1) * sne_ref[...]
    yn = y * sc_ref[...] + sh_ref[...]
    o_ref[...] = ein_ref[...] + jnp.maximum(yn, 0.0)


def _e_apply(e_in, enew, sne, scale, shift):
    blk = BE_BLK
    return pl.pallas_call(
        _e_apply_body,
        grid=(N_EDGES // blk,),
        in_specs=[
            pl.BlockSpec((blk, HID), lambda i: (i, 0)),
            pl.BlockSpec((2, blk, HALF), lambda i: (0, i, 0)),
            pl.BlockSpec((blk, 1), lambda i: (i, 0)),
            pl.BlockSpec((1, HID), lambda i: (0, 0)),
            pl.BlockSpec((1, HID), lambda i: (0, 0)),
        ],
        out_specs=pl.BlockSpec((blk, HID), lambda i: (i, 0)),
        out_shape=jax.ShapeDtypeStruct((N_EDGES, HID), jnp.float32),
    )(e_in, enew, sne, scale.reshape(1, HID), shift.reshape(1, HID))


SUP = 10
NSUP = N_CHUNKS // SUP


def _sc_mid_body(ce, t1, eh, src4, dst4, zeros,
                 enew, acc_out,
                 src_sv, dst_sv, de0, de1, ev0, ev1, cv0, cv1, sm2,
                 acc, sem_g0, sem_g1, sem_o0, sem_o1):
    c = lax.axis_index("c")
    s = lax.axis_index("s")
    coff = c * HALF
    de = (de0, de1)
    ev = (ev0, ev1)
    cv = (cv0, cv1)
    sem_g = (sem_g0, sem_g1)
    sem_o = (sem_o0, sem_o1)

    row0 = s * NPW
    pltpu.sync_copy(zeros, acc.at[pl.ds(row0, NPW)])
    plsc.subcore_barrier()

    base = s * EPW

    def load_idx(u):
        us = lax.rem(u, 2)
        pltpu.sync_copy(src4.at[s, u], src_sv.at[us])
        pltpu.sync_copy(dst4.at[s, u], dst_sv.at[us])

    def gather_refs(t, p, par):
        u = t // SUP
        us = lax.rem(u, 2)
        jj = lax.rem(t, SUP)
        jjj = jj // 2
        b = base + t * K_CH
        return (
            (t1.at[c].at[src_sv.at[us, jj]], de[p]),
            (eh.at[dst_sv.at[us, jjj, pl.ds(par * K_CH, K_CH)]], ev[p]),
            (ce.at[c, pl.ds(b, K_CH)], cv[p]),
        )

    def issue_chunk(t, p, par):
        for sref, dref in gather_refs(t, p, par):
            pltpu.async_copy(sref, dref, sem_g[p])

    def wait_chunk(t, p, par):
        for sref, dref in gather_refs(t, p, par):
            pltpu.make_async_copy(sref, dref, sem_g[p]).wait()

    def issue_out(t, p):
        b = base + t * K_CH
        pltpu.async_copy(cv[p], enew.at[c, pl.ds(b, K_CH)], sem_o[p])

    def wait_out(t, p):
        b = base + t * K_CH
        pltpu.make_async_copy(cv[p], enew.at[c, pl.ds(b, K_CH)], sem_o[p]).wait()

    def scatter_pair(t, p):
        u = t // SUP
        us = lax.rem(u, 2)
        jjj = lax.rem(t, SUP) // 2
        pltpu.sync_copy(sm2, acc.at[dst_sv.at[us, jjj]], add=True)

    def compute(p, par):
        off = par * K_CH

        def row(i, carry):
            for l in range(HALF // 16):
                sl_h = pl.ds(l * 16, 16)
                sl_e = pl.ds(coff + l * 16, 16)
                sl_b = pl.ds(HALF + l * 16, 16)
                x = cv[p][i, sl_h] + de[p][i, sl_h] + ev[p][i, sl_e]
                sg = 1.0 / (1.0 + jnp.exp(-x))
                cv[p][i, sl_h] = x
                sm2[off + i, sl_h] = sg * de[p][i, sl_b]
                sm2[off + i, sl_b] = sg
            return carry

        lax.fori_loop(0, K_CH, row, 0)

    load_idx(0)
    issue_chunk(0, 0, 0)

    def pair(i, carry):
        for par in range(2):
            t = 2 * i + par
            p = par
            q = 1 - par
            tn = t + 1
            wait_chunk(t, p, par)

            @pl.when(t >= 2)
            def _():
                wait_out(t - 2, p)

            @pl.when(jnp.logical_and(tn < N_CHUNKS, lax.rem(tn, SUP) == 0))
            def _():
                load_idx(tn // SUP)

            @pl.when(tn < N_CHUNKS)
            def _():
                issue_chunk(tn, q, 1 - par)

            compute(p, par)
            issue_out(t, p)
            if par == 1:
                scatter_pair(t, p)
        return carry

    lax.fori_loop(0, N_CHUNKS // 2, pair, 0)
    wait_out(N_CHUNKS - 2, 0)
    wait_out(N_CHUNKS - 1, 1)
    plsc.subcore_barrier()

    pltpu.sync_copy(acc.at[pl.ds(row0, NPW)], acc_out.at[c, pl.ds(row0, NPW)])


_sc_mid = pl.kernel(
    _sc_mid_body,
    out_type=(
        jax.ShapeDtypeStruct((2, N_EDGES, HALF), jnp.float32),
        jax.ShapeDtypeStruct((2, N_PAD, HID), jnp.float32),
    ),
    mesh=plsc.VectorSubcoreMesh(core_axis_name="c", subcore_axis_name="s"),
    scratch_types=[
        pltpu.VMEM((2, SUP, K_CH), jnp.int32),
        pltpu.VMEM((2, SUP // 2, 2 * K_CH), jnp.int32),
        pltpu.VMEM((K_CH, HID), jnp.float32),
        pltpu.VMEM((K_CH, HID), jnp.float32),
        pltpu.VMEM((K_CH, HID), jnp.float32),
        pltpu.VMEM((K_CH, HID), jnp.float32),
        pltpu.VMEM((K_CH, HALF), jnp.float32),
        pltpu.VMEM((K_CH, HALF), jnp.float32),
        pltpu.VMEM((2 * K_CH, HID), jnp.float32),
        pltpu.VMEM_SHARED((N_PAD, HID), jnp.float32),
        pltpu.SemaphoreType.DMA,
        pltpu.SemaphoreType.DMA,
        pltpu.SemaphoreType.DMA,
        pltpu.SemaphoreType.DMA,
    ],
)


def kernel(h, e, edge_index, snorm_n, snorm_e, emb_h_W, emb_h_b, emb_e_W,
           emb_e_b, lin_W, lin_b, bn_scale, bn_bias, mlp_W, mlp_b):
    src4 = edge_index[0].reshape(SC_SUBCORES, NSUP, SUP, K_CH)
    dst4 = edge_index[1].reshape(SC_SUBCORES, NSUP, SUP // 2, 2 * K_CH)
    snn = snorm_n.reshape(N_NODES, 1)
    sne = snorm_e.reshape(N_EDGES, 1)
    zeros_stripe = jnp.zeros((NPW, HID), jnp.float32)

    h = _mm(h, emb_h_W, emb_h_b, BN_BLK)
    e = _mm(e, emb_e_W, emb_e_b, BE_BLK)

    for l in range(L):
        h_in, e_in = h, e
        ah, t1, eh = _node_mm(h, lin_W[l], lin_b[l])
        ce2 = _edge_mm(e, lin_W[l, 2], lin_b[l, 2])

        enew2, acc2 = _sc_mid(ce2, t1, eh, src4, dst4, zeros_stripe)

        h = _h_post(ah, acc2, h_in, snn, bn_scale[l, 0], bn_bias[l, 0])

        stats = _e_stats(enew2, sne)
        mean = stats[0] / N_EDGES
        var = stats[1] / N_EDGES - mean * mean
        scale = bn_scale[l, 1] * jax.lax.rsqrt(var + EPS_BN)
        shift = bn_bias[l, 1] - mean * scale
        e = _e_apply(e_in, enew2, sne, scale, shift)

    w_pad = jnp.zeros((HID, HID), jnp.float32).at[:, :N_CLASSES].set(mlp_W)
    b_pad = jnp.zeros((HID,), jnp.float32).at[:N_CLASSES].set(mlp_b)
    out = _mm(h, w_pad, b_pad, BN_BLK)
    return out[:, :N_CLASSES]


# speedup vs baseline: 2.0206x; 1.0491x over previous
import functools

import jax
import jax.numpy as jnp
from jax import lax
from jax.experimental import pallas as pl
from jax.experimental.pallas import tpu as pltpu
from jax.experimental.pallas import tpu_sc as plsc

N_NODES = 10000
N_EDGES = 320000
HID = 128
HALF = 64
L = 4
N_CLASSES = 10
EPS_BN = 1e-5
EPS_DIV = 1e-6

BN_BLK = 2000
BE_BLK = 4000

SC_CORES = 2
SC_SUBCORES = 16
EPW = N_EDGES // SC_SUBCORES
K_CH = 40
N_CHUNKS = EPW // K_CH
N_PAD = 10240
NPW = N_PAD // SC_SUBCORES


def _mm_body(x_ref, w_ref, b_ref, o_ref):
    o_ref[...] = (
        jnp.dot(x_ref[...], w_ref[...], preferred_element_type=jnp.float32)
        + b_ref[...]
    )


def _mm(x, w, b, blk):
    rows = x.shape[0]
    return pl.pallas_call(
        _mm_body,
        grid=(rows // blk,),
        in_specs=[
            pl.BlockSpec((blk, HID), lambda i: (i, 0)),
            pl.BlockSpec((HID, HID), lambda i: (0, 0)),
            pl.BlockSpec((1, HID), lambda i: (0, 0)),
        ],
        out_specs=pl.BlockSpec((blk, HID), lambda i: (i, 0)),
        out_shape=jax.ShapeDtypeStruct((rows, HID), jnp.float32),
    )(x, w, b.reshape(1, HID))


def _node_mm_body(h_ref, w_ref, b_ref, ah_ref, t1_ref, eh_ref):
    h = h_ref[...]
    ah_ref[...] = jnp.dot(h, w_ref[0], preferred_element_type=jnp.float32) + b_ref[0, 0]
    eh_ref[...] = jnp.dot(h, w_ref[4], preferred_element_type=jnp.float32) + b_ref[0, 4]
    bh = jnp.dot(h, w_ref[1], preferred_element_type=jnp.float32) + b_ref[0, 1]
    dh = jnp.dot(h, w_ref[3], preferred_element_type=jnp.float32) + b_ref[0, 3]
    t1_ref[0] = jnp.concatenate([dh[:, :HALF], bh[:, :HALF]], axis=1)
    t1_ref[1] = jnp.concatenate([dh[:, HALF:], bh[:, HALF:]], axis=1)


def _node_mm(h, w5, b5):
    blk = BN_BLK
    spec = pl.BlockSpec((blk, HID), lambda i: (i, 0))
    shape = jax.ShapeDtypeStruct((N_NODES, HID), jnp.float32)
    return pl.pallas_call(
        _node_mm_body,
        grid=(N_NODES // blk,),
        in_specs=[
            spec,
            pl.BlockSpec((5, HID, HID), lambda i: (0, 0, 0)),
            pl.BlockSpec((1, 5, HID), lambda i: (0, 0, 0)),
        ],
        out_specs=[
            spec,
            pl.BlockSpec((2, blk, HID), lambda i: (0, i, 0)),
            spec,
        ],
        out_shape=[
            shape,
            jax.ShapeDtypeStruct((2, N_NODES, HID), jnp.float32),
            shape,
        ],
    )(h, w5, b5.reshape(1, 5, HID))


def _edge_mm_body(e_ref, w_ref, b_ref, o_ref):
    y = jnp.dot(e_ref[...], w_ref[...], preferred_element_type=jnp.float32) + b_ref[...]
    o_ref[0] = y[:, :HALF]
    o_ref[1] = y[:, HALF:]


def _edge_mm(e, w, b):
    blk = BE_BLK
    return pl.pallas_call(
        _edge_mm_body,
        grid=(N_EDGES // blk,),
        in_specs=[
            pl.BlockSpec((blk, HID), lambda i: (i, 0)),
            pl.BlockSpec((HID, HID), lambda i: (0, 0)),
            pl.BlockSpec((1, HID), lambda i: (0, 0)),
        ],
        out_specs=pl.BlockSpec((2, blk, HALF), lambda i: (0, i, 0)),
        out_shape=jax.ShapeDtypeStruct((2, N_EDGES, HALF), jnp.float32),
    )(e, w, b.reshape(1, HID))


def _h_post_body(ah_ref, acc_ref, hin_ref, snn_ref, g_ref, bta_ref, o_ref):
    a0 = acc_ref[0][:N_NODES]
    a1 = acc_ref[1][:N_NODES]
    ssh = jnp.concatenate([a0[:, :HALF], a1[:, :HALF]], axis=1)
    ss = jnp.concatenate([a0[:, HALF:], a1[:, HALF:]], axis=1)
    hn = ah_ref[...] + ssh / (ss + EPS_DIV)
    y = hn * snn_ref[...]
    mean = jnp.mean(y, axis=0, keepdims=True)
    d = y - mean
    var = jnp.mean(d * d, axis=0, keepdims=True)
    yn = g_ref[...] * d * jax.lax.rsqrt(var + EPS_BN) + bta_ref[...]
    o_ref[...] = hin_ref[...] + jnp.maximum(yn, 0.0)


def _h_post(ah, acc, h_in, snn, gamma, beta):
    return pl.pallas_call(
        _h_post_body,
        out_shape=jax.ShapeDtypeStruct((N_NODES, HID), jnp.float32),
    )(ah, acc, h_in, snn, gamma.reshape(1, HID), beta.reshape(1, HID))


def _e_stats_body(en_ref, sne_ref, o_ref, acc_ref):
    i = pl.program_id(0)

    @pl.when(i == 0)
    def _():
        acc_ref[...] = jnp.zeros_like(acc_ref)

    y = jnp.concatenate([en_ref[0], en_ref[1]], axis=1) * sne_ref[...]
    s1 = jnp.sum(y, axis=0)
    s2 = jnp.sum(y * y, axis=0)
    acc_ref[0, :] += s1
    acc_ref[1, :] += s2

    @pl.when(i == pl.num_programs(0) - 1)
    def _():
        o_ref[...] = acc_ref[...]


def _e_stats(enew, sne):
    blk = BE_BLK
    return pl.pallas_call(
        _e_stats_body,
        grid=(N_EDGES // blk,),
        in_specs=[
            pl.BlockSpec((2, blk, HALF), lambda i: (0, i, 0)),
            pl.BlockSpec((blk, 1), lambda i: (i, 0)),
        ],
        out_specs=pl.BlockSpec((2, HID), lambda i: (0, 0)),
        out_shape=jax.ShapeDtypeStruct((2, HID), jnp.float32),
        scratch_shapes=[pltpu.VMEM((2, HID), jnp.float32)],
    )(enew, sne)


def _e_apply_body(ein_ref, en_ref, sne_ref, sc_ref, sh_ref, o_ref):
    y = jnp.concatenate([en_ref[0], en_ref[1]], axis=1) * sne_ref[...]
    yn = y * sc_ref[...] + sh_ref[...]
    o_ref[...] = ein_ref[...] + jnp.maximum(yn, 0.0)


def _e_apply(e_in, enew, sne, scale, shift):
    blk = BE_BLK
    return pl.pallas_call(
        _e_apply_body,
        grid=(N_EDGES // blk,),
        in_specs=[
            pl.BlockSpec((blk, HID), lambda i: (i, 0)),
            pl.BlockSpec((2, blk, HALF), lambda i: (0, i, 0)),
            pl.BlockSpec((blk, 1), lambda i: (i, 0)),
            pl.BlockSpec((1, HID), lambda i: (0, 0)),
            pl.BlockSpec((1, HID), lambda i: (0, 0)),
        ],
        out_specs=pl.BlockSpec((blk, HID), lambda i: (i, 0)),
        out_shape=jax.ShapeDtypeStruct((N_EDGES, HID), jnp.float32),
    )(e_in, enew, sne, scale.reshape(1, HID), shift.reshape(1, HID))


SUP = 10
NSUP = N_CHUNKS // SUP


def _sc_mid_body(ce, t1, eh, src4, dst4, zeros,
                 enew, acc_out,
                 src_sv, dst_sv, de0, de1, ev0, ev1, cv0, cv1, sm0, sm1,
                 acc, sem_g0, sem_g1, sem_o0, sem_o1, sem_s0, sem_s1):
    c = lax.axis_index("c")
    s = lax.axis_index("s")
    coff = c * HALF
    de = (de0, de1)
    ev = (ev0, ev1)
    cv = (cv0, cv1)
    sm = (sm0, sm1)
    sem_g = (sem_g0, sem_g1)
    sem_o = (sem_o0, sem_o1)
    sem_s = (sem_s0, sem_s1)

    row0 = s * NPW
    pltpu.sync_copy(zeros, acc.at[pl.ds(row0, NPW)])
    plsc.subcore_barrier()

    base = s * EPW

    def load_idx(u):
        us = lax.rem(u, 2)
        pltpu.sync_copy(src4.at[s, u], src_sv.at[us])
        pltpu.sync_copy(dst4.at[s, u], dst_sv.at[us])

    def gather_refs(t, p, par):
        u = t // SUP
        us = lax.rem(u, 2)
        jj = lax.rem(t, SUP)
        jjj = jj // 2
        b = base + t * K_CH
        return (
            (t1.at[c].at[src_sv.at[us, jj]], de[p]),
            (eh.at[dst_sv.at[us, jjj, pl.ds(par * K_CH, K_CH)]], ev[p]),
            (ce.at[c, pl.ds(b, K_CH)], cv[p]),
        )

    def issue_chunk(t, p, par):
        for sref, dref in gather_refs(t, p, par):
            pltpu.async_copy(sref, dref, sem_g[p])

    def wait_chunk(t, p, par):
        for sref, dref in gather_refs(t, p, par):
            pltpu.make_async_copy(sref, dref, sem_g[p]).wait()

    def issue_out(t, p):
        b = base + t * K_CH
        pltpu.async_copy(cv[p], enew.at[c, pl.ds(b, K_CH)], sem_o[p])

    def wait_out(t, p):
        b = base + t * K_CH
        pltpu.make_async_copy(cv[p], enew.at[c, pl.ds(b, K_CH)], sem_o[p]).wait()

    def scatter_ref(t, par):
        u = t // SUP
        us = lax.rem(u, 2)
        jjj = lax.rem(t, SUP) // 2
        return acc.at[dst_sv.at[us, jjj, pl.ds(par * K_CH, K_CH)]]

    def issue_scatter(t, p, par):
        pltpu.async_copy(sm[p], scatter_ref(t, par), sem_s[p], add=True)

    def wait_scatter(t, p, par):
        pltpu.make_async_copy(sm[p], scatter_ref(t, par), sem_s[p]).wait()

    def compute(p, par):
        def row(i, carry):
            for l in range(HALF // 16):
                sl_h = pl.ds(l * 16, 16)
                sl_e = pl.ds(coff + l * 16, 16)
                sl_b = pl.ds(HALF + l * 16, 16)
                x = cv[p][i, sl_h] + de[p][i, sl_h] + ev[p][i, sl_e]
                sg = 1.0 / (1.0 + jnp.exp(-x))
                cv[p][i, sl_h] = x
                sm[p][i, sl_h] = sg * de[p][i, sl_b]
                sm[p][i, sl_b] = sg
            return carry

        lax.fori_loop(0, K_CH, row, 0)

    load_idx(0)
    issue_chunk(0, 0, 0)

    def pair(i, carry):
        for par in range(2):
            t = 2 * i + par
            p = par
            q = 1 - par
            tn = t + 1
            wait_chunk(t, p, par)

            @pl.when(t >= 2)
            def _():
                wait_out(t - 2, p)
                wait_scatter(t - 2, p, par)

            @pl.when(jnp.logical_and(tn < N_CHUNKS, lax.rem(tn, SUP) == 0))
            def _():
                load_idx(tn // SUP)

            @pl.when(tn < N_CHUNKS)
            def _():
                issue_chunk(tn, q, 1 - par)

            compute(p, par)
            issue_out(t, p)
            issue_scatter(t, p, par)
        return carry

    lax.fori_loop(0, N_CHUNKS // 2, pair, 0)
    wait_out(N_CHUNKS - 2, 0)
    wait_out(N_CHUNKS - 1, 1)
    wait_scatter(N_CHUNKS - 2, 0, 0)
    wait_scatter(N_CHUNKS - 1, 1, 1)
    plsc.subcore_barrier()

    pltpu.sync_copy(acc.at[pl.ds(row0, NPW)], acc_out.at[c, pl.ds(row0, NPW)])


_sc_mid = pl.kernel(
    _sc_mid_body,
    out_type=(
        jax.ShapeDtypeStruct((2, N_EDGES, HALF), jnp.float32),
        jax.ShapeDtypeStruct((2, N_PAD, HID), jnp.float32),
    ),
    mesh=plsc.VectorSubcoreMesh(core_axis_name="c", subcore_axis_name="s"),
    scratch_types=[
        pltpu.VMEM((2, SUP, K_CH), jnp.int32),
        pltpu.VMEM((2, SUP // 2, 2 * K_CH), jnp.int32),
        pltpu.VMEM((K_CH, HID), jnp.float32),
        pltpu.VMEM((K_CH, HID), jnp.float32),
        pltpu.VMEM((K_CH, HID), jnp.float32),
        pltpu.VMEM((K_CH, HID), jnp.float32),
        pltpu.VMEM((K_CH, HALF), jnp.float32),
        pltpu.VMEM((K_CH, HALF), jnp.float32),
        pltpu.VMEM((K_CH, HID), jnp.float32),
        pltpu.VMEM((K_CH, HID), jnp.float32),
        pltpu.VMEM_SHARED((N_PAD, HID), jnp.float32),
        pltpu.SemaphoreType.DMA,
        pltpu.SemaphoreType.DMA,
        pltpu.SemaphoreType.DMA,
        pltpu.SemaphoreType.DMA,
        pltpu.SemaphoreType.DMA,
        pltpu.SemaphoreType.DMA,
    ],
)


def kernel(h, e, edge_index, snorm_n, snorm_e, emb_h_W, emb_h_b, emb_e_W,
           emb_e_b, lin_W, lin_b, bn_scale, bn_bias, mlp_W, mlp_b):
    src4 = edge_index[0].reshape(SC_SUBCORES, NSUP, SUP, K_CH)
    dst4 = edge_index[1].reshape(SC_SUBCORES, NSUP, SUP // 2, 2 * K_CH)
    snn = snorm_n.reshape(N_NODES, 1)
    sne = snorm_e.reshape(N_EDGES, 1)
    zeros_stripe = jnp.zeros((NPW, HID), jnp.float32)

    h = _mm(h, emb_h_W, emb_h_b, BN_BLK)
    e = _mm(e, emb_e_W, emb_e_b, BE_BLK)

    for l in range(L):
        h_in, e_in = h, e
        ah, t1, eh = _node_mm(h, lin_W[l], lin_b[l])
        ce2 = _edge_mm(e, lin_W[l, 2], lin_b[l, 2])

        enew2, acc2 = _sc_mid(ce2, t1, eh, src4, dst4, zeros_stripe)

        h = _h_post(ah, acc2, h_in, snn, bn_scale[l, 0], bn_bias[l, 0])

        stats = _e_stats(enew2, sne)
        mean = stats[0] / N_EDGES
        var = stats[1] / N_EDGES - mean * mean
        scale = bn_scale[l, 1] * jax.lax.rsqrt(var + EPS_BN)
        shift = bn_bias[l, 1] - mean * scale
        e = _e_apply(e_in, enew2, sne, scale, shift)

    w_pad = jnp.zeros((HID, HID), jnp.float32).at[:, :N_CLASSES].set(mlp_W)
    b_pad = jnp.zeros((HID,), jnp.float32).at[:N_CLASSES].set(mlp_b)
    out = _mm(h, w_pad, b_pad, BN_BLK)
    return out[:, :N_CLASSES]


# speedup vs baseline: 3.1046x; 1.5365x over previous
import functools

import jax
import jax.numpy as jnp
from jax import lax
from jax.experimental import pallas as pl
from jax.experimental.pallas import tpu as pltpu
from jax.experimental.pallas import tpu_sc as plsc

N_NODES = 10000
N_EDGES = 320000
HID = 128
HALF = 64
L = 4
N_CLASSES = 10
EPS_BN = 1e-5
EPS_DIV = 1e-6

BN_BLK = 2000
BE_BLK = 4000

SC_CORES = 2
SC_SUBCORES = 16
EPW = N_EDGES // SC_SUBCORES
K_CH = 40
N_CHUNKS = EPW // K_CH
N_PAD = 10240
NPW = N_PAD // SC_SUBCORES


def _mm_body(x_ref, w_ref, b_ref, o_ref):
    o_ref[...] = (
        jnp.dot(x_ref[...], w_ref[...], preferred_element_type=jnp.float32)
        + b_ref[...]
    )


def _mm(x, w, b, blk):
    rows = x.shape[0]
    return pl.pallas_call(
        _mm_body,
        grid=(rows // blk,),
        in_specs=[
            pl.BlockSpec((blk, HID), lambda i: (i, 0)),
            pl.BlockSpec((HID, HID), lambda i: (0, 0)),
            pl.BlockSpec((1, HID), lambda i: (0, 0)),
        ],
        out_specs=pl.BlockSpec((blk, HID), lambda i: (i, 0)),
        out_shape=jax.ShapeDtypeStruct((rows, HID), jnp.float32),
    )(x, w, b.reshape(1, HID))


def _node_mm_body(h_ref, w_ref, b_ref, ah_ref, t1_ref, eh_ref):
    h = h_ref[...]
    ah_ref[...] = jnp.dot(h, w_ref[0], preferred_element_type=jnp.float32) + b_ref[0, 0]
    eh_ref[...] = jnp.dot(h, w_ref[4], preferred_element_type=jnp.float32) + b_ref[0, 4]
    bh = jnp.dot(h, w_ref[1], preferred_element_type=jnp.float32) + b_ref[0, 1]
    dh = jnp.dot(h, w_ref[3], preferred_element_type=jnp.float32) + b_ref[0, 3]
    t1_ref[0] = jnp.concatenate([dh[:, :HALF], bh[:, :HALF]], axis=1)
    t1_ref[1] = jnp.concatenate([dh[:, HALF:], bh[:, HALF:]], axis=1)


def _node_mm(h, w5, b5):
    blk = BN_BLK
    spec = pl.BlockSpec((blk, HID), lambda i: (i, 0))
    shape = jax.ShapeDtypeStruct((N_NODES, HID), jnp.float32)
    return pl.pallas_call(
        _node_mm_body,
        grid=(N_NODES // blk,),
        in_specs=[
            spec,
            pl.BlockSpec((5, HID, HID), lambda i: (0, 0, 0)),
            pl.BlockSpec((1, 5, HID), lambda i: (0, 0, 0)),
        ],
        out_specs=[
            spec,
            pl.BlockSpec((2, blk, HID), lambda i: (0, i, 0)),
            spec,
        ],
        out_shape=[
            shape,
            jax.ShapeDtypeStruct((2, N_NODES, HID), jnp.float32),
            shape,
        ],
    )(h, w5, b5.reshape(1, 5, HID))


def _edge_mm_body(e_ref, w_ref, b_ref, o_ref):
    y = jnp.dot(e_ref[...], w_ref[...], preferred_element_type=jnp.float32) + b_ref[...]
    o_ref[0] = y[:, :HALF]
    o_ref[1] = y[:, HALF:]


def _edge_mm(e, w, b):
    blk = BE_BLK
    return pl.pallas_call(
        _edge_mm_body,
        grid=(N_EDGES // blk,),
        in_specs=[
            pl.BlockSpec((blk, HID), lambda i: (i, 0)),
            pl.BlockSpec((HID, HID), lambda i: (0, 0)),
            pl.BlockSpec((1, HID), lambda i: (0, 0)),
        ],
        out_specs=pl.BlockSpec((2, blk, HALF), lambda i: (0, i, 0)),
        out_shape=jax.ShapeDtypeStruct((2, N_EDGES, HALF), jnp.float32),
    )(e, w, b.reshape(1, HID))


def _h_post_body(ah_ref, acc_ref, hin_ref, snn_ref, g_ref, bta_ref, o_ref):
    a0 = acc_ref[0][:N_NODES]
    a1 = acc_ref[1][:N_NODES]
    ssh = jnp.concatenate([a0[:, :HALF], a1[:, :HALF]], axis=1)
    ss = jnp.concatenate([a0[:, HALF:], a1[:, HALF:]], axis=1)
    hn = ah_ref[...] + ssh / (ss + EPS_DIV)
    y = hn * snn_ref[...]
    mean = jnp.mean(y, axis=0, keepdims=True)
    d = y - mean
    var = jnp.mean(d * d, axis=0, keepdims=True)
    yn = g_ref[...] * d * jax.lax.rsqrt(var + EPS_BN) + bta_ref[...]
    o_ref[...] = hin_ref[...] + jnp.maximum(yn, 0.0)


def _h_post(ah, acc, h_in, snn, gamma, beta):
    return pl.pallas_call(
        _h_post_body,
        out_shape=jax.ShapeDtypeStruct((N_NODES, HID), jnp.float32),
    )(ah, acc, h_in, snn, gamma.reshape(1, HID), beta.reshape(1, HID))


def _e_stats_body(en_ref, sne_ref, o_ref, acc_ref):
    i = pl.program_id(0)

    @pl.when(i == 0)
    def _():
        acc_ref[...] = jnp.zeros_like(acc_ref)

    y = jnp.concatenate([en_ref[0], en_ref[1]], axis=1) * sne_ref[...]
    s1 = jnp.sum(y, axis=0)
    s2 = jnp.sum(y * y, axis=0)
    acc_ref[0, :] += s1
    acc_ref[1, :] += s2

    @pl.when(i == pl.num_programs(0) - 1)
    def _():
        o_ref[...] = acc_ref[...]


def _e_stats(enew, sne):
    blk = BE_BLK
    return pl.pallas_call(
        _e_stats_body,
        grid=(N_EDGES // blk,),
        in_specs=[
            pl.BlockSpec((2, blk, HALF), lambda i: (0, i, 0)),
            pl.BlockSpec((blk, 1), lambda i: (i, 0)),
        ],
        out_specs=pl.BlockSpec((2, HID), lambda i: (0, 0)),
        out_shape=jax.ShapeDtypeStruct((2, HID), jnp.float32),
        scratch_shapes=[pltpu.VMEM((2, HID), jnp.float32)],
    )(enew, sne)


def _e_apply_body(ein_ref, en_ref, sne_ref, sc_ref, sh_ref, o_ref):
    y = jnp.concatenate([en_ref[0], en_ref[1]], axis=1) * sne_ref[...]
    yn = y * sc_ref[...] + sh_ref[...]
    o_ref[...] = ein_ref[...] + jnp.maximum(yn, 0.0)


def _e_apply(e_in, enew, sne, scale, shift):
    blk = BE_BLK
    return pl.pallas_call(
        _e_apply_body,
        grid=(N_EDGES // blk,),
        in_specs=[
            pl.BlockSpec((blk, HID), lambda i: (i, 0)),
            pl.BlockSpec((2, blk, HALF), lambda i: (0, i, 0)),
            pl.BlockSpec((blk, 1), lambda i: (i, 0)),
            pl.BlockSpec((1, HID), lambda i: (0, 0)),
            pl.BlockSpec((1, HID), lambda i: (0, 0)),
        ],
        out_specs=pl.BlockSpec((blk, HID), lambda i: (i, 0)),
        out_shape=jax.ShapeDtypeStruct((N_EDGES, HID), jnp.float32),
    )(e_in, enew, sne, scale.reshape(1, HID), shift.reshape(1, HID))


SUP = 10
NSUP = N_CHUNKS // SUP


def _sc_mid_body(ce, t1, eh, src4, dst4, zeros,
                 enew, acc_out,
                 src_sv, dst_sv, de0, de1, ev0, ev1, cv0, cv1, sm0, sm1,
                 acc, sem_g0, sem_g1, sem_o0, sem_o1, sem_s0, sem_s1):
    c = lax.axis_index("c")
    s = lax.axis_index("s")
    coff = c * HALF
    de = (de0, de1)
    ev = (ev0, ev1)
    cv = (cv0, cv1)
    sm = (sm0, sm1)
    sem_g = (sem_g0, sem_g1)
    sem_o = (sem_o0, sem_o1)
    sem_s = (sem_s0, sem_s1)

    row0 = s * NPW
    pltpu.sync_copy(zeros, acc.at[pl.ds(row0, NPW)])
    plsc.subcore_barrier()

    base = s * EPW

    def load_idx(u):
        us = lax.rem(u, 2)
        pltpu.sync_copy(src4.at[s, u], src_sv.at[us])
        pltpu.sync_copy(dst4.at[s, u], dst_sv.at[us])

    def gather_refs(t, p, par):
        u = t // SUP
        us = lax.rem(u, 2)
        jj = lax.rem(t, SUP)
        jjj = jj // 2
        b = base + t * K_CH
        return (
            (t1.at[c].at[src_sv.at[us, jj]], de[p]),
            (eh.at[dst_sv.at[us, jjj, pl.ds(par * K_CH, K_CH)]], ev[p]),
            (ce.at[c, pl.ds(b, K_CH)], cv[p]),
        )

    def issue_chunk(t, p, par):
        for sref, dref in gather_refs(t, p, par):
            pltpu.async_copy(sref, dref, sem_g[p])

    def wait_chunk(t, p, par):
        for sref, dref in gather_refs(t, p, par):
            pltpu.make_async_copy(sref, dref, sem_g[p]).wait()

    def issue_out(t, p):
        b = base + t * K_CH
        pltpu.async_copy(cv[p], enew.at[c, pl.ds(b, K_CH)], sem_o[p])

    def wait_out(t, p):
        b = base + t * K_CH
        pltpu.make_async_copy(cv[p], enew.at[c, pl.ds(b, K_CH)], sem_o[p]).wait()

    def scatter_ref(t, par):
        u = t // SUP
        us = lax.rem(u, 2)
        jjj = lax.rem(t, SUP) // 2
        return acc.at[dst_sv.at[us, jjj, pl.ds(par * K_CH, K_CH)]]

    def issue_scatter(t, p, par):
        pltpu.async_copy(sm[p], scatter_ref(t, par), sem_s[p], add=True)

    def wait_scatter(t, p, par):
        pltpu.make_async_copy(sm[p], scatter_ref(t, par), sem_s[p]).wait()

    def compute(p, par):
        def rowpair(i2, carry):
            rs = pl.ds(2 * i2, 2)
            for l in range(HALF // 16):
                sl_h = pl.ds(l * 16, 16)
                sl_e = pl.ds(coff + l * 16, 16)
                sl_b = pl.ds(HALF + l * 16, 16)
                x = cv[p][rs, sl_h] + de[p][rs, sl_h] + ev[p][rs, sl_e]
                sg = 1.0 / (1.0 + jnp.exp(-x))
                cv[p][rs, sl_h] = x
                sm[p][rs, sl_h] = sg * de[p][rs, sl_b]
                sm[p][rs, sl_b] = sg
            return carry

        lax.fori_loop(0, K_CH // 2, rowpair, 0)

    load_idx(0)
    issue_chunk(0, 0, 0)

    def pair(i, carry):
        for par in range(2):
            t = 2 * i + par
            p = par
            q = 1 - par
            tn = t + 1
            wait_chunk(t, p, par)

            @pl.when(t >= 2)
            def _():
                wait_out(t - 2, p)
                wait_scatter(t - 2, p, par)

            @pl.when(jnp.logical_and(tn < N_CHUNKS, lax.rem(tn, SUP) == 0))
            def _():
                load_idx(tn // SUP)

            @pl.when(tn < N_CHUNKS)
            def _():
                issue_chunk(tn, q, 1 - par)

            compute(p, par)
            issue_out(t, p)
            issue_scatter(t, p, par)
        return carry

    lax.fori_loop(0, N_CHUNKS // 2, pair, 0)
    wait_out(N_CHUNKS - 2, 0)
    wait_out(N_CHUNKS - 1, 1)
    wait_scatter(N_CHUNKS - 2, 0, 0)
    wait_scatter(N_CHUNKS - 1, 1, 1)
    plsc.subcore_barrier()

    pltpu.sync_copy(acc.at[pl.ds(row0, NPW)], acc_out.at[c, pl.ds(row0, NPW)])


_sc_mid = pl.kernel(
    _sc_mid_body,
    out_type=(
        jax.ShapeDtypeStruct((2, N_EDGES, HALF), jnp.float32),
        jax.ShapeDtypeStruct((2, N_PAD, HID), jnp.float32),
    ),
    mesh=plsc.VectorSubcoreMesh(core_axis_name="c", subcore_axis_name="s"),
    scratch_types=[
        pltpu.VMEM((2, SUP, K_CH), jnp.int32),
        pltpu.VMEM((2, SUP // 2, 2 * K_CH), jnp.int32),
        pltpu.VMEM((K_CH, HID), jnp.float32),
        pltpu.VMEM((K_CH, HID), jnp.float32),
        pltpu.VMEM((K_CH, HID), jnp.float32),
        pltpu.VMEM((K_CH, HID), jnp.float32),
        pltpu.VMEM((K_CH, HALF), jnp.float32),
        pltpu.VMEM((K_CH, HALF), jnp.float32),
        pltpu.VMEM((K_CH, HID), jnp.float32),
        pltpu.VMEM((K_CH, HID), jnp.float32),
        pltpu.VMEM_SHARED((N_PAD, HID), jnp.float32),
        pltpu.SemaphoreType.DMA,
        pltpu.SemaphoreType.DMA,
        pltpu.SemaphoreType.DMA,
        pltpu.SemaphoreType.DMA,
        pltpu.SemaphoreType.DMA,
        pltpu.SemaphoreType.DMA,
    ],
)


def kernel(h, e, edge_index, snorm_n, snorm_e, emb_h_W, emb_h_b, emb_e_W,
           emb_e_b, lin_W, lin_b, bn_scale, bn_bias, mlp_W, mlp_b):
    src4 = edge_index[0].reshape(SC_SUBCORES, NSUP, SUP, K_CH)
    dst4 = edge_index[1].reshape(SC_SUBCORES, NSUP, SUP // 2, 2 * K_CH)
    snn = snorm_n.reshape(N_NODES, 1)
    sne = snorm_e.reshape(N_EDGES, 1)
    zeros_stripe = jnp.zeros((NPW, HID), jnp.float32)

    h = _mm(h, emb_h_W, emb_h_b, BN_BLK)
    e = _mm(e, emb_e_W, emb_e_b, BE_BLK)

    for l in range(L):
        h_in, e_in = h, e
        ah, t1, eh = _node_mm(h, lin_W[l], lin_b[l])
        ce2 = _edge_mm(e, lin_W[l, 2], lin_b[l, 2])

        enew2, acc2 = _sc_mid(ce2, t1, eh, src4, dst4, zeros_stripe)

        h = _h_post(ah, acc2, h_in, snn, bn_scale[l, 0], bn_bias[l, 0])

        stats = _e_stats(enew2, sne)
        mean = stats[0] / N_EDGES
        var = stats[1] / N_EDGES - mean * mean
        scale = bn_scale[l, 1] * jax.lax.rsqrt(var + EPS_BN)
        shift = bn_bias[l, 1] - mean * scale
        e = _e_apply(e_in, enew2, sne, scale, shift)

    w_pad = jnp.zeros((HID, HID), jnp.float32).at[:, :N_CLASSES].set(mlp_W)
    b_pad = jnp.zeros((HID,), jnp.float32).at[:N_CLASSES].set(mlp_b)
    out = _mm(h, w_pad, b_pad, BN_BLK)
    return out[:, :N_CLASSES]


# speedup vs baseline: 3.4387x; 1.1076x over previous
import functools

import jax
import jax.numpy as jnp
from jax import lax
from jax.experimental import pallas as pl
from jax.experimental.pallas import tpu as pltpu
from jax.experimental.pallas import tpu_sc as plsc

N_NODES = 10000
N_EDGES = 320000
HID = 128
HALF = 64
L = 4
N_CLASSES = 10
EPS_BN = 1e-5
EPS_DIV = 1e-6

BN_BLK = 2000
BE_BLK = 4000

SC_CORES = 2
SC_SUBCORES = 16
EPW = N_EDGES // SC_SUBCORES
K_CH = 40
N_CHUNKS = EPW // K_CH
N_PAD = 10240
NPW = N_PAD // SC_SUBCORES


def _mm_body(x_ref, w_ref, b_ref, o_ref):
    o_ref[...] = (
        jnp.dot(x_ref[...], w_ref[...], preferred_element_type=jnp.float32)
        + b_ref[...]
    )


def _mm(x, w, b, blk):
    rows = x.shape[0]
    return pl.pallas_call(
        _mm_body,
        grid=(rows // blk,),
        in_specs=[
            pl.BlockSpec((blk, HID), lambda i: (i, 0)),
            pl.BlockSpec((HID, HID), lambda i: (0, 0)),
            pl.BlockSpec((1, HID), lambda i: (0, 0)),
        ],
        out_specs=pl.BlockSpec((blk, HID), lambda i: (i, 0)),
        out_shape=jax.ShapeDtypeStruct((rows, HID), jnp.float32),
    )(x, w, b.reshape(1, HID))


def _node_mm_body(h_ref, w_ref, b_ref, ah_ref, t1_ref, eh_ref):
    h = h_ref[...]
    ah_ref[...] = jnp.dot(h, w_ref[0], preferred_element_type=jnp.float32) + b_ref[0, 0]
    eh_ref[...] = jnp.dot(h, w_ref[4], preferred_element_type=jnp.float32) + b_ref[0, 4]
    bh = jnp.dot(h, w_ref[1], preferred_element_type=jnp.float32) + b_ref[0, 1]
    dh = jnp.dot(h, w_ref[3], preferred_element_type=jnp.float32) + b_ref[0, 3]
    t1_ref[0] = jnp.concatenate([dh[:, :HALF], bh[:, :HALF]], axis=1)
    t1_ref[1] = jnp.concatenate([dh[:, HALF:], bh[:, HALF:]], axis=1)


def _node_mm(h, w5, b5):
    blk = BN_BLK
    spec = pl.BlockSpec((blk, HID), lambda i: (i, 0))
    shape = jax.ShapeDtypeStruct((N_NODES, HID), jnp.float32)
    return pl.pallas_call(
        _node_mm_body,
        grid=(N_NODES // blk,),
        in_specs=[
            spec,
            pl.BlockSpec((5, HID, HID), lambda i: (0, 0, 0)),
            pl.BlockSpec((1, 5, HID), lambda i: (0, 0, 0)),
        ],
        out_specs=[
            spec,
            pl.BlockSpec((2, blk, HID), lambda i: (0, i, 0)),
            spec,
        ],
        out_shape=[
            shape,
            jax.ShapeDtypeStruct((2, N_NODES, HID), jnp.float32),
            shape,
        ],
    )(h, w5, b5.reshape(1, 5, HID))


def _edge_mm_body(e_ref, w_ref, b_ref, o_ref):
    y = jnp.dot(e_ref[...], w_ref[...], preferred_element_type=jnp.float32) + b_ref[...]
    o_ref[0] = y[:, :HALF]
    o_ref[1] = y[:, HALF:]


def _edge_mm(e, w, b):
    blk = BE_BLK
    return pl.pallas_call(
        _edge_mm_body,
        grid=(N_EDGES // blk,),
        in_specs=[
            pl.BlockSpec((blk, HID), lambda i: (i, 0)),
            pl.BlockSpec((HID, HID), lambda i: (0, 0)),
            pl.BlockSpec((1, HID), lambda i: (0, 0)),
        ],
        out_specs=pl.BlockSpec((2, blk, HALF), lambda i: (0, i, 0)),
        out_shape=jax.ShapeDtypeStruct((2, N_EDGES, HALF), jnp.float32),
    )(e, w, b.reshape(1, HID))


def _h_post_body(ah_ref, acc_ref, hin_ref, snn_ref, g_ref, bta_ref, o_ref):
    a0 = acc_ref[0][:N_NODES]
    a1 = acc_ref[1][:N_NODES]
    ssh = jnp.concatenate([a0[:, :HALF], a1[:, :HALF]], axis=1)
    ss = jnp.concatenate([a0[:, HALF:], a1[:, HALF:]], axis=1)
    hn = ah_ref[...] + ssh / (ss + EPS_DIV)
    y = hn * snn_ref[...]
    mean = jnp.mean(y, axis=0, keepdims=True)
    d = y - mean
    var = jnp.mean(d * d, axis=0, keepdims=True)
    yn = g_ref[...] * d * jax.lax.rsqrt(var + EPS_BN) + bta_ref[...]
    o_ref[...] = hin_ref[...] + jnp.maximum(yn, 0.0)


def _h_post(ah, acc, h_in, snn, gamma, beta):
    return pl.pallas_call(
        _h_post_body,
        out_shape=jax.ShapeDtypeStruct((N_NODES, HID), jnp.float32),
    )(ah, acc, h_in, snn, gamma.reshape(1, HID), beta.reshape(1, HID))


def _e_stats_body(en_ref, sne_ref, o_ref, acc_ref):
    i = pl.program_id(0)

    @pl.when(i == 0)
    def _():
        acc_ref[...] = jnp.zeros_like(acc_ref)

    y = jnp.concatenate([en_ref[0], en_ref[1]], axis=1) * sne_ref[...]
    s1 = jnp.sum(y, axis=0)
    s2 = jnp.sum(y * y, axis=0)
    acc_ref[0, :] += s1
    acc_ref[1, :] += s2

    @pl.when(i == pl.num_programs(0) - 1)
    def _():
        o_ref[...] = acc_ref[...]


def _e_stats(enew, sne):
    blk = BE_BLK
    return pl.pallas_call(
        _e_stats_body,
        grid=(N_EDGES // blk,),
        in_specs=[
            pl.BlockSpec((2, blk, HALF), lambda i: (0, i, 0)),
            pl.BlockSpec((blk, 1), lambda i: (i, 0)),
        ],
        out_specs=pl.BlockSpec((2, HID), lambda i: (0, 0)),
        out_shape=jax.ShapeDtypeStruct((2, HID), jnp.float32),
        scratch_shapes=[pltpu.VMEM((2, HID), jnp.float32)],
    )(enew, sne)


def _e_apply_body(ein_ref, en_ref, sne_ref, sc_ref, sh_ref, o_ref):
    y = jnp.concatenate([en_ref[0], en_ref[1]], axis=1) * sne_ref[...]
    yn = y * sc_ref[...] + sh_ref[...]
    o_ref[...] = ein_ref[...] + jnp.maximum(yn, 0.0)


def _e_apply(e_in, enew, sne, scale, shift):
    blk = BE_BLK
    return pl.pallas_call(
        _e_apply_body,
        grid=(N_EDGES // blk,),
        in_specs=[
            pl.BlockSpec((blk, HID), lambda i: (i, 0)),
            pl.BlockSpec((2, blk, HALF), lambda i: (0, i, 0)),
            pl.BlockSpec((blk, 1), lambda i: (i, 0)),
            pl.BlockSpec((1, HID), lambda i: (0, 0)),
            pl.BlockSpec((1, HID), lambda i: (0, 0)),
        ],
        out_specs=pl.BlockSpec((blk, HID), lambda i: (i, 0)),
        out_shape=jax.ShapeDtypeStruct((N_EDGES, HID), jnp.float32),
    )(e_in, enew, sne, scale.reshape(1, HID), shift.reshape(1, HID))


SUP = 10
NSUP = N_CHUNKS // SUP


def _sc_mid_body(ce, t1, eh, src4, dst4, zeros,
                 enew, acc_out,
                 src_sv, dst_sv, de0, de1, ev0, ev1, cv0, cv1, sm0, sm1,
                 acc, sem_g0, sem_g1, sem_o0, sem_o1, sem_s0, sem_s1):
    c = lax.axis_index("c")
    s = lax.axis_index("s")
    coff = c * HALF
    de = (de0, de1)
    ev = (ev0, ev1)
    cv = (cv0, cv1)
    sm = (sm0, sm1)
    sem_g = (sem_g0, sem_g1)
    sem_o = (sem_o0, sem_o1)
    sem_s = (sem_s0, sem_s1)

    row0 = s * NPW
    pltpu.sync_copy(zeros, acc.at[pl.ds(row0, NPW)])
    plsc.subcore_barrier()

    base = s * EPW

    def load_idx(u):
        us = lax.rem(u, 2)
        pltpu.sync_copy(src4.at[s, u], src_sv.at[us])
        pltpu.sync_copy(dst4.at[s, u], dst_sv.at[us])

    def gather_refs(t, p, par):
        u = t // SUP
        us = lax.rem(u, 2)
        jj = lax.rem(t, SUP)
        jjj = jj // 2
        b = base + t * K_CH
        return (
            (t1.at[c].at[src_sv.at[us, jj]], de[p]),
            (eh.at[dst_sv.at[us, jjj, pl.ds(par * K_CH, K_CH)]], ev[p]),
            (ce.at[c, pl.ds(b, K_CH)], cv[p]),
        )

    def issue_chunk(t, p, par):
        for sref, dref in gather_refs(t, p, par):
            pltpu.async_copy(sref, dref, sem_g[p])

    def wait_chunk(t, p, par):
        for sref, dref in gather_refs(t, p, par):
            pltpu.make_async_copy(sref, dref, sem_g[p]).wait()

    def issue_out(t, p):
        b = base + t * K_CH
        pltpu.async_copy(cv[p], enew.at[c, pl.ds(b, K_CH)], sem_o[p])

    def wait_out(t, p):
        b = base + t * K_CH
        pltpu.make_async_copy(cv[p], enew.at[c, pl.ds(b, K_CH)], sem_o[p]).wait()

    def scatter_ref(t, par):
        u = t // SUP
        us = lax.rem(u, 2)
        jjj = lax.rem(t, SUP) // 2
        return acc.at[dst_sv.at[us, jjj, pl.ds(par * K_CH, K_CH)]]

    def issue_scatter(t, p, par):
        pltpu.async_copy(sm[p], scatter_ref(t, par), sem_s[p], add=True)

    def wait_scatter(t, p, par):
        pltpu.make_async_copy(sm[p], scatter_ref(t, par), sem_s[p]).wait()

    def compute(p, par):
        def rowpair(i2, carry):
            rs = pl.ds(8 * i2, 8)
            for l in range(HALF // 16):
                sl_h = pl.ds(l * 16, 16)
                sl_e = pl.ds(coff + l * 16, 16)
                sl_b = pl.ds(HALF + l * 16, 16)
                x = cv[p][rs, sl_h] + de[p][rs, sl_h] + ev[p][rs, sl_e]
                sg = 1.0 / (1.0 + jnp.exp(-x))
                cv[p][rs, sl_h] = x
                sm[p][rs, sl_h] = sg * de[p][rs, sl_b]
                sm[p][rs, sl_b] = sg
            return carry

        lax.fori_loop(0, K_CH // 8, rowpair, 0)

    load_idx(0)
    issue_chunk(0, 0, 0)

    def pair(i, carry):
        for par in range(2):
            t = 2 * i + par
            p = par
            q = 1 - par
            tn = t + 1
            wait_chunk(t, p, par)

            @pl.when(t >= 2)
            def _():
                wait_out(t - 2, p)
                wait_scatter(t - 2, p, par)

            @pl.when(jnp.logical_and(tn < N_CHUNKS, lax.rem(tn, SUP) == 0))
            def _():
                load_idx(tn // SUP)

            @pl.when(tn < N_CHUNKS)
            def _():
                issue_chunk(tn, q, 1 - par)

            compute(p, par)
            issue_out(t, p)
            issue_scatter(t, p, par)
        return carry

    lax.fori_loop(0, N_CHUNKS // 2, pair, 0)
    wait_out(N_CHUNKS - 2, 0)
    wait_out(N_CHUNKS - 1, 1)
    wait_scatter(N_CHUNKS - 2, 0, 0)
    wait_scatter(N_CHUNKS - 1, 1, 1)
    plsc.subcore_barrier()

    pltpu.sync_copy(acc.at[pl.ds(row0, NPW)], acc_out.at[c, pl.ds(row0, NPW)])


_sc_mid = pl.kernel(
    _sc_mid_body,
    out_type=(
        jax.ShapeDtypeStruct((2, N_EDGES, HALF), jnp.float32),
        jax.ShapeDtypeStruct((2, N_PAD, HID), jnp.float32),
    ),
    mesh=plsc.VectorSubcoreMesh(core_axis_name="c", subcore_axis_name="s"),
    scratch_types=[
        pltpu.VMEM((2, SUP, K_CH), jnp.int32),
        pltpu.VMEM((2, SUP // 2, 2 * K_CH), jnp.int32),
        pltpu.VMEM((K_CH, HID), jnp.float32),
        pltpu.VMEM((K_CH, HID), jnp.float32),
        pltpu.VMEM((K_CH, HID), jnp.float32),
        pltpu.VMEM((K_CH, HID), jnp.float32),
        pltpu.VMEM((K_CH, HALF), jnp.float32),
        pltpu.VMEM((K_CH, HALF), jnp.float32),
        pltpu.VMEM((K_CH, HID), jnp.float32),
        pltpu.VMEM((K_CH, HID), jnp.float32),
        pltpu.VMEM_SHARED((N_PAD, HID), jnp.float32),
        pltpu.SemaphoreType.DMA,
        pltpu.SemaphoreType.DMA,
        pltpu.SemaphoreType.DMA,
        pltpu.SemaphoreType.DMA,
        pltpu.SemaphoreType.DMA,
        pltpu.SemaphoreType.DMA,
    ],
)


def kernel(h, e, edge_index, snorm_n, snorm_e, emb_h_W, emb_h_b, emb_e_W,
           emb_e_b, lin_W, lin_b, bn_scale, bn_bias, mlp_W, mlp_b):
    src4 = edge_index[0].reshape(SC_SUBCORES, NSUP, SUP, K_CH)
    dst4 = edge_index[1].reshape(SC_SUBCORES, NSUP, SUP // 2, 2 * K_CH)
    snn = snorm_n.reshape(N_NODES, 1)
    sne = snorm_e.reshape(N_EDGES, 1)
    zeros_stripe = jnp.zeros((NPW, HID), jnp.float32)

    h = _mm(h, emb_h_W, emb_h_b, BN_BLK)
    e = _mm(e, emb_e_W, emb_e_b, BE_BLK)

    for l in range(L):
        h_in, e_in = h, e
        ah, t1, eh = _node_mm(h, lin_W[l], lin_b[l])
        ce2 = _edge_mm(e, lin_W[l, 2], lin_b[l, 2])

        enew2, acc2 = _sc_mid(ce2, t1, eh, src4, dst4, zeros_stripe)

        h = _h_post(ah, acc2, h_in, snn, bn_scale[l, 0], bn_bias[l, 0])

        stats = _e_stats(enew2, sne)
        mean = stats[0] / N_EDGES
        var = stats[1] / N_EDGES - mean * mean
        scale = bn_scale[l, 1] * jax.lax.rsqrt(var + EPS_BN)
        shift = bn_bias[l, 1] - mean * scale
        e = _e_apply(e_in, enew2, sne, scale, shift)

    w_pad = jnp.zeros((HID, HID), jnp.float32).at[:, :N_CLASSES].set(mlp_W)
    b_pad = jnp.zeros((HID,), jnp.float32).at[:N_CLASSES].set(mlp_b)
    out = _mm(h, w_pad, b_pad, BN_BLK)
    return out[:, :N_CLASSES]


# speedup vs baseline: 3.4471x; 1.0024x over previous
import functools

import jax
import jax.numpy as jnp
from jax import lax
from jax.experimental import pallas as pl
from jax.experimental.pallas import tpu as pltpu
from jax.experimental.pallas import tpu_sc as plsc

N_NODES = 10000
N_EDGES = 320000
HID = 128
HALF = 64
L = 4
N_CLASSES = 10
EPS_BN = 1e-5
EPS_DIV = 1e-6

BN_BLK = 2000
BE_BLK = 4000

SC_CORES = 2
SC_SUBCORES = 16
EPW = N_EDGES // SC_SUBCORES
K_CH = 40
N_CHUNKS = EPW // K_CH
N_PAD = 10240
NPW = N_PAD // SC_SUBCORES


def _mm_body(x_ref, w_ref, b_ref, o_ref):
    o_ref[...] = (
        jnp.dot(x_ref[...], w_ref[...], preferred_element_type=jnp.float32)
        + b_ref[...]
    )


def _mm(x, w, b, blk):
    rows = x.shape[0]
    return pl.pallas_call(
        _mm_body,
        grid=(rows // blk,),
        in_specs=[
            pl.BlockSpec((blk, HID), lambda i: (i, 0)),
            pl.BlockSpec((HID, HID), lambda i: (0, 0)),
            pl.BlockSpec((1, HID), lambda i: (0, 0)),
        ],
        out_specs=pl.BlockSpec((blk, HID), lambda i: (i, 0)),
        out_shape=jax.ShapeDtypeStruct((rows, HID), jnp.float32),
    )(x, w, b.reshape(1, HID))


def _node_mm_body(h_ref, w_ref, b_ref, ah_ref, t1_ref, eh_ref):
    h = h_ref[...]
    ah_ref[...] = jnp.dot(h, w_ref[0], preferred_element_type=jnp.float32) + b_ref[0, 0]
    eh_ref[...] = jnp.dot(h, w_ref[4], preferred_element_type=jnp.float32) + b_ref[0, 4]
    bh = jnp.dot(h, w_ref[1], preferred_element_type=jnp.float32) + b_ref[0, 1]
    dh = jnp.dot(h, w_ref[3], preferred_element_type=jnp.float32) + b_ref[0, 3]
    t1_ref[0] = jnp.concatenate([dh[:, :HALF], bh[:, :HALF]], axis=1)
    t1_ref[1] = jnp.concatenate([dh[:, HALF:], bh[:, HALF:]], axis=1)


def _node_mm(h, w5, b5):
    blk = BN_BLK
    spec = pl.BlockSpec((blk, HID), lambda i: (i, 0))
    shape = jax.ShapeDtypeStruct((N_NODES, HID), jnp.float32)
    return pl.pallas_call(
        _node_mm_body,
        grid=(N_NODES // blk,),
        in_specs=[
            spec,
            pl.BlockSpec((5, HID, HID), lambda i: (0, 0, 0)),
            pl.BlockSpec((1, 5, HID), lambda i: (0, 0, 0)),
        ],
        out_specs=[
            spec,
            pl.BlockSpec((2, blk, HID), lambda i: (0, i, 0)),
            spec,
        ],
        out_shape=[
            shape,
            jax.ShapeDtypeStruct((2, N_NODES, HID), jnp.float32),
            shape,
        ],
    )(h, w5, b5.reshape(1, 5, HID))


def _edge_mm_body(e_ref, w_ref, b_ref, o_ref):
    y = jnp.dot(e_ref[...], w_ref[...], preferred_element_type=jnp.float32) + b_ref[...]
    o_ref[0] = y[:, :HALF]
    o_ref[1] = y[:, HALF:]


def _edge_mm(e, w, b):
    blk = BE_BLK
    return pl.pallas_call(
        _edge_mm_body,
        grid=(N_EDGES // blk,),
        in_specs=[
            pl.BlockSpec((blk, HID), lambda i: (i, 0)),
            pl.BlockSpec((HID, HID), lambda i: (0, 0)),
            pl.BlockSpec((1, HID), lambda i: (0, 0)),
        ],
        out_specs=pl.BlockSpec((2, blk, HALF), lambda i: (0, i, 0)),
        out_shape=jax.ShapeDtypeStruct((2, N_EDGES, HALF), jnp.float32),
    )(e, w, b.reshape(1, HID))


def _h_post_body(ah_ref, acc_ref, hin_ref, snn_ref, g_ref, bta_ref, o_ref):
    a0 = acc_ref[0][:N_NODES]
    a1 = acc_ref[1][:N_NODES]
    ssh = jnp.concatenate([a0[:, :HALF], a1[:, :HALF]], axis=1)
    ss = jnp.concatenate([a0[:, HALF:], a1[:, HALF:]], axis=1)
    hn = ah_ref[...] + ssh / (ss + EPS_DIV)
    y = hn * snn_ref[...]
    mean = jnp.mean(y, axis=0, keepdims=True)
    d = y - mean
    var = jnp.mean(d * d, axis=0, keepdims=True)
    yn = g_ref[...] * d * jax.lax.rsqrt(var + EPS_BN) + bta_ref[...]
    o_ref[...] = hin_ref[...] + jnp.maximum(yn, 0.0)


def _h_post(ah, acc, h_in, snn, gamma, beta):
    return pl.pallas_call(
        _h_post_body,
        out_shape=jax.ShapeDtypeStruct((N_NODES, HID), jnp.float32),
    )(ah, acc, h_in, snn, gamma.reshape(1, HID), beta.reshape(1, HID))


def _e_stats_body(en_ref, sne_ref, o_ref, acc_ref):
    i = pl.program_id(0)

    @pl.when(i == 0)
    def _():
        acc_ref[...] = jnp.zeros_like(acc_ref)

    y = jnp.concatenate([en_ref[0], en_ref[1]], axis=1) * sne_ref[...]
    s1 = jnp.sum(y, axis=0)
    s2 = jnp.sum(y * y, axis=0)
    acc_ref[0, :] += s1
    acc_ref[1, :] += s2

    @pl.when(i == pl.num_programs(0) - 1)
    def _():
        o_ref[...] = acc_ref[...]


def _e_stats(enew, sne):
    blk = BE_BLK
    return pl.pallas_call(
        _e_stats_body,
        grid=(N_EDGES // blk,),
        in_specs=[
            pl.BlockSpec((2, blk, HALF), lambda i: (0, i, 0)),
            pl.BlockSpec((blk, 1), lambda i: (i, 0)),
        ],
        out_specs=pl.BlockSpec((2, HID), lambda i: (0, 0)),
        out_shape=jax.ShapeDtypeStruct((2, HID), jnp.float32),
        scratch_shapes=[pltpu.VMEM((2, HID), jnp.float32)],
    )(enew, sne)


def _e_apply_body(ein_ref, en_ref, sne_ref, sc_ref, sh_ref, o_ref):
    y = jnp.concatenate([en_ref[0], en_ref[1]], axis=1) * sne_ref[...]
    yn = y * sc_ref[...] + sh_ref[...]
    o_ref[...] = ein_ref[...] + jnp.maximum(yn, 0.0)


def _e_apply(e_in, enew, sne, scale, shift):
    blk = BE_BLK
    return pl.pallas_call(
        _e_apply_body,
        grid=(N_EDGES // blk,),
        in_specs=[
            pl.BlockSpec((blk, HID), lambda i: (i, 0)),
            pl.BlockSpec((2, blk, HALF), lambda i: (0, i, 0)),
            pl.BlockSpec((blk, 1), lambda i: (i, 0)),
            pl.BlockSpec((1, HID), lambda i: (0, 0)),
            pl.BlockSpec((1, HID), lambda i: (0, 0)),
        ],
        out_specs=pl.BlockSpec((blk, HID), lambda i: (i, 0)),
        out_shape=jax.ShapeDtypeStruct((N_EDGES, HID), jnp.float32),
    )(e_in, enew, sne, scale.reshape(1, HID), shift.reshape(1, HID))


SUP = 10
NSUP = N_CHUNKS // SUP


def _sc_mid_body(ce, t1, eh, src4, dst4, zeros,
                 enew, acc_out,
                 src_sv, dst_sv, de0, de1, ev0, ev1, cv0, cv1, sm0, sm1,
                 acc, sem_g0, sem_g1, sem_o0, sem_o1, sem_s0, sem_s1):
    c = lax.axis_index("c")
    s = lax.axis_index("s")
    coff = c * HALF
    de = (de0, de1)
    ev = (ev0, ev1)
    cv = (cv0, cv1)
    sm = (sm0, sm1)
    sem_g = (sem_g0, sem_g1)
    sem_o = (sem_o0, sem_o1)
    sem_s = (sem_s0, sem_s1)

    row0 = s * NPW
    pltpu.sync_copy(zeros, acc.at[pl.ds(row0, NPW)])
    plsc.subcore_barrier()

    base = s * EPW

    def load_idx(u):
        us = lax.rem(u, 2)
        pltpu.sync_copy(src4.at[s, u], src_sv.at[us])
        pltpu.sync_copy(dst4.at[s, u], dst_sv.at[us])

    def gather_refs(t, p, par):
        u = t // SUP
        us = lax.rem(u, 2)
        jj = lax.rem(t, SUP)
        jjj = jj // 2
        b = base + t * K_CH
        return (
            (t1.at[c].at[src_sv.at[us, jj]], de[p]),
            (eh.at[dst_sv.at[us, jjj, pl.ds(par * K_CH, K_CH)]], ev[p]),
            (ce.at[c, pl.ds(b, K_CH)], cv[p]),
        )

    def issue_chunk(t, p, par):
        for sref, dref in gather_refs(t, p, par):
            pltpu.async_copy(sref, dref, sem_g[p])

    def wait_chunk(t, p, par):
        for sref, dref in gather_refs(t, p, par):
            pltpu.make_async_copy(sref, dref, sem_g[p]).wait()

    def issue_out(t, p):
        b = base + t * K_CH
        pltpu.async_copy(cv[p], enew.at[c, pl.ds(b, K_CH)], sem_o[p])

    def wait_out(t, p):
        b = base + t * K_CH
        pltpu.make_async_copy(cv[p], enew.at[c, pl.ds(b, K_CH)], sem_o[p]).wait()

    def scatter_ref(t, par):
        u = t // SUP
        us = lax.rem(u, 2)
        jjj = lax.rem(t, SUP) // 2
        return acc.at[dst_sv.at[us, jjj, pl.ds(par * K_CH, K_CH)]]

    def issue_scatter(t, p, par):
        pltpu.async_copy(sm[p], scatter_ref(t, par), sem_s[p], add=True)

    def wait_scatter(t, p, par):
        pltpu.make_async_copy(sm[p], scatter_ref(t, par), sem_s[p]).wait()

    def compute(p, par):
        def rowpair(i2, carry):
            rs = pl.ds(8 * i2, 8)
            for l in range(HALF // 16):
                sl_h = pl.ds(l * 16, 16)
                sl_e = pl.ds(coff + l * 16, 16)
                sl_b = pl.ds(HALF + l * 16, 16)
                x = cv[p][rs, sl_h] + de[p][rs, sl_h] + ev[p][rs, sl_e]
                sg = 1.0 / (1.0 + jnp.exp(-x))
                cv[p][rs, sl_h] = x
                sm[p][rs, sl_h] = sg * de[p][rs, sl_b]
                sm[p][rs, sl_b] = sg
            return carry

        lax.fori_loop(0, K_CH // 8, rowpair, 0)

    load_idx(0)
    issue_chunk(0, 0, 0)

    def pair(i, carry):
        for par in range(2):
            t = 2 * i + par
            p = par
            q = 1 - par
            tn = t + 1
            wait_chunk(t, p, par)

            @pl.when(t >= 2)
            def _():
                wait_out(t - 2, p)
                wait_scatter(t - 2, p, par)

            @pl.when(jnp.logical_and(tn < N_CHUNKS, lax.rem(tn, SUP) == 0))
            def _():
                load_idx(tn // SUP)

            @pl.when(tn < N_CHUNKS)
            def _():
                issue_chunk(tn, q, 1 - par)

            compute(p, par)
            issue_out(t, p)
            issue_scatter(t, p, par)
        return carry

    lax.fori_loop(0, N_CHUNKS // 2, pair, 0)
    wait_out(N_CHUNKS - 2, 0)
    wait_out(N_CHUNKS - 1, 1)
    wait_scatter(N_CHUNKS - 2, 0, 0)
    wait_scatter(N_CHUNKS - 1, 1, 1)
    plsc.subcore_barrier()

    pltpu.sync_copy(acc.at[pl.ds(row0, NPW)], acc_out.at[c, pl.ds(row0, NPW)])


_sc_mid = pl.kernel(
    _sc_mid_body,
    out_type=(
        jax.ShapeDtypeStruct((2, N_EDGES, HALF), jnp.float32),
        jax.ShapeDtypeStruct((2, N_PAD, HID), jnp.float32),
    ),
    mesh=plsc.VectorSubcoreMesh(core_axis_name="c", subcore_axis_name="s"),
    scratch_types=[
        pltpu.VMEM((2, SUP, K_CH), jnp.int32),
        pltpu.VMEM((2, SUP // 2, 2 * K_CH), jnp.int32),
        pltpu.VMEM((K_CH, HID), jnp.float32),
        pltpu.VMEM((K_CH, HID), jnp.float32),
        pltpu.VMEM((K_CH, HID), jnp.float32),
        pltpu.VMEM((K_CH, HID), jnp.float32),
        pltpu.VMEM((K_CH, HALF), jnp.float32),
        pltpu.VMEM((K_CH, HALF), jnp.float32),
        pltpu.VMEM((K_CH, HID), jnp.float32),
        pltpu.VMEM((K_CH, HID), jnp.float32),
        pltpu.VMEM_SHARED((N_PAD, HID), jnp.float32),
        pltpu.SemaphoreType.DMA,
        pltpu.SemaphoreType.DMA,
        pltpu.SemaphoreType.DMA,
        pltpu.SemaphoreType.DMA,
        pltpu.SemaphoreType.DMA,
        pltpu.SemaphoreType.DMA,
    ],
)


def kernel(h, e, edge_index, snorm_n, snorm_e, emb_h_W, emb_h_b, emb_e_W,
           emb_e_b, lin_W, lin_b, bn_scale, bn_bias, mlp_W, mlp_b):
    src4 = edge_index[0].reshape(SC_SUBCORES, NSUP, SUP, K_CH)
    dst4 = edge_index[1].reshape(SC_SUBCORES, NSUP, SUP // 2, 2 * K_CH)
    snn = snorm_n.reshape(N_NODES, 1)
    sne = snorm_e.reshape(N_EDGES, 1)
    zeros_stripe = jnp.zeros((NPW, HID), jnp.float32)

    h = _mm(h, emb_h_W, emb_h_b, BN_BLK)
    e = _mm(e, emb_e_W, emb_e_b, BE_BLK)

    for l in range(L):
        h_in, e_in = h, e
        ah, t1, eh = _node_mm(h, lin_W[l], lin_b[l])
        ce2 = _edge_mm(e, lin_W[l, 2], lin_b[l, 2])

        enew2, acc2 = _sc_mid(ce2, t1, eh, src4, dst4, zeros_stripe)

        h = _h_post(ah, acc2, h_in, snn, bn_scale[l, 0], bn_bias[l, 0])

        if l < L - 1:
            stats = _e_stats(enew2, sne)
            mean = stats[0] / N_EDGES
            var = stats[1] / N_EDGES - mean * mean
            scale = bn_scale[l, 1] * jax.lax.rsqrt(var + EPS_BN)
            shift = bn_bias[l, 1] - mean * scale
            e = _e_apply(e_in, enew2, sne, scale, shift)

    w_pad = jnp.zeros((HID, HID), jnp.float32).at[:, :N_CLASSES].set(mlp_W)
    b_pad = jnp.zeros((HID,), jnp.float32).at[:N_CLASSES].set(mlp_b)
    out = _mm(h, w_pad, b_pad, BN_BLK)
    return out[:, :N_CLASSES]
